# baseline (speedup 1.0000x reference)
import functools

import numpy as np
import jax
import jax.numpy as jnp
from jax import lax
from jax.experimental import pallas as pl
from jax.experimental.pallas import tpu as pltpu
from jax.experimental.pallas import tpu_sc as plsc

N = 10000
E = 320000
D = 128
C = 16

NC = 2
NS = 16
NW = NC * NS
K = 512
PER_W = 10240
EPAD = NW * PER_W
NCHUNK = PER_W // K
NPAD = 10112
RPT = NPAD // NS
PADROWS = NPAD - N

_f32 = jnp.float32
_i32 = jnp.int32

_SC_MESH = plsc.VectorSubcoreMesh(core_axis_name="c", subcore_axis_name="s")

_PERM = np.array([(p % 8) * 8 + p // 8 for p in range(64)])
_ORIG = np.array([(p % 8) * 8 + p // 8 for p in range(64)])


def _tc_prep12_body(x_ref, W1_ref, B1s_ref, B1d_ref, W2_ref, B2s_ref, B2d_ref,
                    H1_ref, S1_ref, D1_ref, X1_ref, H2_ref, S2_ref, D2_ref, X2_ref):
    xb = x_ref[...]
    for (W_ref, Bs_ref, Bd_ref, H_ref, S_ref, D_ref, X_ref) in (
        (W1_ref, B1s_ref, B1d_ref, H1_ref, S1_ref, D1_ref, X1_ref),
        (W2_ref, B2s_ref, B2d_ref, H2_ref, S2_ref, D2_ref, X2_ref),
    ):
        h = jnp.dot(xb, W_ref[...], preferred_element_type=_f32)
        a_s = jnp.dot(h, Bs_ref[...], preferred_element_type=_f32)
        a_d = jnp.dot(h, Bd_ref[...], preferred_element_type=_f32)
        H_ref[...] = h
        S_ref[...] = jnp.concatenate([a_s, a_s], axis=1)
        D_ref[...] = jnp.concatenate([a_d, a_d], axis=1)
        t = a_s + a_d
        X_ref[...] = jnp.exp(jnp.maximum(t, 0.2 * t))


def _tc_prep12(x, W1p, B1s, B1d, W2p, B2s, B2d):
    R = 1000
    grid = (N // R,)
    row = lambda i: (i, 0)
    const = lambda i: (0, 0)
    out16 = jax.ShapeDtypeStruct((N, 16), _f32)
    out8 = jax.ShapeDtypeStruct((N, 8), _f32)
    return pl.pallas_call(
        _tc_prep12_body,
        grid=grid,
        in_specs=[
            pl.BlockSpec((R, D), row),
            pl.BlockSpec((D, 64), const), pl.BlockSpec((64, 8), const), pl.BlockSpec((64, 8), const),
            pl.BlockSpec((D, 64), const), pl.BlockSpec((64, 8), const), pl.BlockSpec((64, 8), const),
        ],
        out_specs=[
            pl.BlockSpec((R, 64), row), pl.BlockSpec((R, 16), row),
            pl.BlockSpec((R, 16), row), pl.BlockSpec((R, 8), row),
            pl.BlockSpec((R, 64), row), pl.BlockSpec((R, 16), row),
            pl.BlockSpec((R, 16), row), pl.BlockSpec((R, 8), row),
        ],
        out_shape=[jax.ShapeDtypeStruct((N, 64), _f32), out16, out16, out8,
                   jax.ShapeDtypeStruct((N, 64), _f32), out16, out16, out8],
    )(x, W1p, B1s, B1d, W2p, B2s, B2d)


@functools.partial(
    pl.kernel,
    out_type=[
        jax.ShapeDtypeStruct((EPAD, 16), _f32),
        jax.ShapeDtypeStruct((EPAD, 16), _f32),
        jax.ShapeDtypeStruct((NC * NPAD, 8), _f32),
        jax.ShapeDtypeStruct((NC * NPAD, 8), _f32),
    ],
    mesh=_SC_MESH,
    compiler_params=pltpu.CompilerParams(use_tc_tiling_on_sc=False, needs_layout_passes=False),
    scratch_types=[
        pltpu.VMEM((K,), _i32), pltpu.VMEM((K,), _i32),
        pltpu.VMEM((K,), _i32), pltpu.VMEM((K,), _i32),
        pltpu.VMEM((K, 16), _f32), pltpu.VMEM((K, 16), _f32),
        pltpu.VMEM((K, 16), _f32), pltpu.VMEM((K, 16), _f32),
        pltpu.VMEM((K, 16), _f32), pltpu.VMEM((K, 16), _f32),
        pltpu.VMEM((K, 8), _f32), pltpu.VMEM((K, 8), _f32),
        pltpu.VMEM((K,), _i32), pltpu.VMEM((K,), _i32),
        pltpu.VMEM_SHARED((NPAD, 8), _f32),
        pltpu.VMEM_SHARED((NPAD, 8), _f32),
        pltpu.SemaphoreType.DMA,
        pltpu.SemaphoreType.DMA,
        pltpu.SemaphoreType.DMA,
        pltpu.SemaphoreType.DMA,
        pltpu.SemaphoreType.DMA,
        pltpu.SemaphoreType.DMA,
    ],
)
def _sc_pass1_12(src1, dst1, src2, dst2, S1, D1, S2, D2,
                 ex1, ex2, s1o, s2o,
                 idx_s0, idx_s1, idx_d0, idx_d1, rows_s0, rows_s1,
                 rows_d0, rows_d1, stage0, stage1, st8_0, st8_1,
                 sidx0, sidx1, acc1, acc2,
                 sem0, sem1, sem_s0, sem_s1, sem_e0, sem_e1):
    cid = lax.axis_index("c")
    sid = lax.axis_index("s")
    wid = sid * NC + cid

    idx_s = (idx_s0, idx_s1)
    idx_d = (idx_d0, idx_d1)
    rows_s = (rows_s0, rows_s1)
    rows_d = (rows_d0, rows_d1)
    stage = (stage0, stage1)
    stage8 = (st8_0, st8_1)
    sidx = (sidx0, sidx1)
    sem = (sem0, sem1)
    sem_s = (sem_s0, sem_s1)
    sem_e = (sem_e0, sem_e1)

    zero16 = jnp.zeros((16,), _f32)
    lane = lax.iota(_i32, 16)
    col8 = lax.rem(lane, 8)
    msk8 = lane < 8

    @pl.loop(0, K // 2)
    def _zero(i):
        plsc.store_scatter(st8_0, [lax.div(lane, 8) + i * 2, col8], zero16)

    for acc in (acc1, acc2):
        pltpu.sync_copy(st8_0, acc.at[pl.ds(sid * RPT, K)])
        pltpu.sync_copy(st8_0.at[pl.ds(0, RPT - K)], acc.at[pl.ds(sid * RPT + K, RPT - K)])
    plsc.subcore_barrier()

    for (src, dst, S, Dt, exo, acc) in ((src1, dst1, S1, D1, ex1, acc1),
                                        (src2, dst2, S2, D2, ex2, acc2)):
        def load_idx(j, b):
            base = wid * PER_W + j * K
            pltpu.sync_copy(src.at[pl.ds(base, K)], idx_s[b])
            pltpu.sync_copy(dst.at[pl.ds(base, K)], idx_d[b])

        def fire(b):
            pltpu.async_copy(S.at[idx_s[b]], rows_s[b], sem[b])
            pltpu.async_copy(Dt.at[idx_d[b]], rows_d[b], sem[b])

        def wait(b):
            pltpu.make_async_copy(S.at[idx_s[b]], rows_s[b], sem[b]).wait()
            pltpu.make_async_copy(Dt.at[idx_d[b]], rows_d[b], sem[b]).wait()

        load_idx(0, 0)
        fire(0)

        @pl.loop(0, NCHUNK // 2)
        def _pair(j2):
            for b in (0, 1):
                j = j2 * 2 + b
                nb = 1 - b
                nxt = j + 1

                @pl.when(nxt < NCHUNK)
                def _prefetch():
                    load_idx(nxt, nb)
                    fire(nb)

                wait(b)

                @pl.when(j >= 2)
                def _drain():
                    base_p = wid * PER_W + (j - 2) * K
                    pltpu.make_async_copy(stage[b], exo.at[pl.ds(base_p, K)], sem_e[b]).wait()
                    pltpu.make_async_copy(stage8[b], acc.at[sidx[b]], sem_s[b]).wait()

                rs_b, rd_b, st_b, s8_b = rows_s[b], rows_d[b], stage[b], stage8[b]

                @pl.loop(0, K, unroll=4)
                def _edge(i):
                    v = rs_b[i, :] + rd_b[i, :]
                    ex = jnp.exp(jnp.maximum(v, 0.2 * v))
                    st_b[i, :] = ex
                    plsc.store_scatter(s8_b, [jnp.full((16,), i, _i32), col8],
                                       ex, mask=msk8)

                base = wid * PER_W + j * K
                id_b, si_b = idx_d[b], sidx[b]

                @pl.loop(0, K // 16)
                def _sicopy(g):
                    si_b[pl.ds(g * 16, 16)] = id_b[pl.ds(g * 16, 16)]

                pltpu.async_copy(st_b, exo.at[pl.ds(base, K)], sem_e[b])
                pltpu.async_copy(s8_b, acc.at[sidx[b]], sem_s[b], add=True)

        for b, j_last in ((NCHUNK % 2, NCHUNK - 2), ((NCHUNK - 1) % 2, NCHUNK - 1)):
            base_p = wid * PER_W + j_last * K
            pltpu.make_async_copy(stage[b], exo.at[pl.ds(base_p, K)], sem_e[b]).wait()
            pltpu.make_async_copy(stage8[b], acc.at[sidx[b]], sem_s[b]).wait()

    plsc.subcore_barrier()
    rs = pl.ds(sid * RPT, RPT)
    pltpu.sync_copy(acc1.at[rs], s1o.at[pl.ds(cid * NPAD + sid * RPT, RPT)])
    pltpu.sync_copy(acc2.at[rs], s2o.at[pl.ds(cid * NPAD + sid * RPT, RPT)])


def _tc_mid_body(s1_ref, X1_ref, s2_ref, X2_ref, r1_ref, a1_ref, r2_ref, a2_ref):
    for (s_ref, X_ref, r_ref, a_ref) in ((s1_ref, X1_ref, r1_ref, a1_ref),
                                         (s2_ref, X2_ref, r2_ref, a2_ref)):
        s = s_ref[...]
        ex_self = X_ref[...]
        tot = s[:NPAD] + s[NPAD:] + ex_self
        r = 1.0 / (tot + 1e-16)
        r_ref[...] = jnp.concatenate([r, r], axis=1)
        a_ref[...] = ex_self * r


def _tc_mid(s1, X1p, s2, X2p):
    out16 = jax.ShapeDtypeStruct((NPAD, 16), _f32)
    out8 = jax.ShapeDtypeStruct((NPAD, 8), _f32)
    return pl.pallas_call(
        _tc_mid_body,
        out_shape=[out16, out8, out16, out8],
    )(s1, X1p, s2, X2p)


K2 = 256
NCHUNK2 = PER_W // K2


@functools.partial(
    pl.kernel,
    out_type=[
        jax.ShapeDtypeStruct((NC * NPAD, 64), _f32),
        jax.ShapeDtypeStruct((NC * NPAD, 64), _f32),
    ],
    mesh=_SC_MESH,
    compiler_params=pltpu.CompilerParams(use_tc_tiling_on_sc=False, needs_layout_passes=False),
    scratch_types=[
        pltpu.VMEM((K2,), _i32), pltpu.VMEM((K2,), _i32),
        pltpu.VMEM((K2,), _i32), pltpu.VMEM((K2,), _i32),
        pltpu.VMEM((K2, 64), _f32), pltpu.VMEM((K2, 64), _f32),
        pltpu.VMEM((K2, 16), _f32), pltpu.VMEM((K2, 16), _f32),
        pltpu.VMEM((K2, 16), _f32), pltpu.VMEM((K2, 16), _f32),
        pltpu.VMEM((K2, 64), _f32), pltpu.VMEM((K2, 64), _f32),
        pltpu.VMEM((K2,), _i32), pltpu.VMEM((K2,), _i32),
        pltpu.VMEM_SHARED((NPAD, 64), _f32),
        pltpu.SemaphoreType.DMA,
        pltpu.SemaphoreType.DMA,
        pltpu.SemaphoreType.DMA,
        pltpu.SemaphoreType.DMA,
    ],
)
def _sc_pass2_12(src1, dst1, src2, dst2, H1, H2, ex1, ex2, r1, r2,
                 o1, o2,
                 idx_s0, idx_s1, idx_d0, idx_d1, hrows0, hrows1,
                 exrows0, exrows1, rrows0, rrows1, stage0, stage1,
                 sidx0, sidx1, acc,
                 sem0, sem1, sem_s0, sem_s1):
    cid = lax.axis_index("c")
    sid = lax.axis_index("s")
    wid = sid * NC + cid

    idx_s = (idx_s0, idx_s1)
    idx_d = (idx_d0, idx_d1)
    hrows = (hrows0, hrows1)
    exrows = (exrows0, exrows1)
    rrows = (rrows0, rrows1)
    stage = (stage0, stage1)
    sidx = (sidx0, sidx1)
    sem = (sem0, sem1)
    sem_s = (sem_s0, sem_s1)

    zero16 = jnp.zeros((16,), _f32)
    rs = pl.ds(sid * RPT, RPT)

    for (src, dst, H, exi, r, oo) in ((src1, dst1, H1, ex1, r1, o1),
                                      (src2, dst2, H2, ex2, r2, o2)):
        def load_idx(j, b):
            base = wid * PER_W + j * K2
            pltpu.sync_copy(src.at[pl.ds(base, K2)], idx_s[b])
            pltpu.sync_copy(dst.at[pl.ds(base, K2)], idx_d[b])

        def fire(j, b):
            base = wid * PER_W + j * K2
            pltpu.async_copy(H.at[idx_s[b]], hrows[b], sem[b])
            pltpu.async_copy(r.at[idx_d[b]], rrows[b], sem[b])
            pltpu.async_copy(exi.at[pl.ds(base, K2)], exrows[b], sem[b])

        def wait(j, b):
            base = wid * PER_W + j * K2
            pltpu.make_async_copy(H.at[idx_s[b]], hrows[b], sem[b]).wait()
            pltpu.make_async_copy(r.at[idx_d[b]], rrows[b], sem[b]).wait()
            pltpu.make_async_copy(exi.at[pl.ds(base, K2)], exrows[b], sem[b]).wait()

        @pl.loop(0, K2)
        def _zero(i):
            for q in range(4):
                stage0[i, pl.ds(q * 16, 16)] = zero16

        pltpu.sync_copy(stage0, acc.at[pl.ds(sid * RPT, K2)])
        pltpu.sync_copy(stage0, acc.at[pl.ds(sid * RPT + K2, K2)])
        pltpu.sync_copy(stage0.at[pl.ds(0, RPT - 2 * K2)],
                        acc.at[pl.ds(sid * RPT + 2 * K2, RPT - 2 * K2)])
        load_idx(0, 0)
        fire(0, 0)
        plsc.subcore_barrier()

        @pl.loop(0, NCHUNK2 // 2)
        def _pair(j2):
            for b in (0, 1):
                j = j2 * 2 + b
                nb = 1 - b
                nxt = j + 1

                @pl.when(nxt < NCHUNK2)
                def _prefetch():
                    load_idx(nxt, nb)
                    fire(nxt, nb)

                wait(j, b)

                @pl.when(j >= 2)
                def _drain():
                    pltpu.make_async_copy(stage[b], acc.at[sidx[b]], sem_s[b]).wait()

                h_b, ex_b, r_b, st_b = hrows[b], exrows[b], rrows[b], stage[b]

                @pl.loop(0, K2, unroll=2)
                def _edge(i):
                    alpha = ex_b[i, :] * r_b[i, :]
                    for q in range(4):
                        st_b[i, pl.ds(q * 16, 16)] = h_b[i, pl.ds(q * 16, 16)] * alpha

                id_b, si_b = idx_d[b], sidx[b]

                @pl.loop(0, K2 // 16)
                def _sicopy(g):
                    si_b[pl.ds(g * 16, 16)] = id_b[pl.ds(g * 16, 16)]

                pltpu.async_copy(st_b, acc.at[sidx[b]], sem_s[b], add=True)

        for b in (NCHUNK2 % 2, (NCHUNK2 - 1) % 2):
            pltpu.make_async_copy(stage[b], acc.at[sidx[b]], sem_s[b]).wait()

        plsc.subcore_barrier()
        pltpu.sync_copy(acc.at[rs], oo.at[pl.ds(cid * NPAD + sid * RPT, RPT)])
        plsc.subcore_barrier()


def _tc_layer3_body(o1a_ref, o1b_ref, H1_ref, a1_ref, b1_ref,
                    o2a_ref, o2b_ref, H2_ref, a2_ref, b2_ref,
                    W3_ref, as3_ref, ad3_ref,
                    H3_ref, A3_ref):
    xs = []
    for (oa, ob, H_ref, a_ref, b_ref) in ((o1a_ref, o1b_ref, H1_ref, a1_ref, b1_ref),
                                          (o2a_ref, o2b_ref, H2_ref, a2_ref, b2_ref)):
        aself = a_ref[...]
        xl = oa[...] + ob[...] + H_ref[...] * jnp.tile(aself, (1, 8)) + b_ref[...]
        xl = jnp.where(xl > 0, xl, jnp.exp(jnp.minimum(xl, 0.0)) - 1.0)
        xs.append(xl)
    xc = jnp.concatenate(xs, axis=1)
    h3 = jnp.dot(xc, W3_ref[...], preferred_element_type=_f32)
    a3s = jnp.sum(h3 * as3_ref[...], axis=1, keepdims=True)
    a3d = jnp.sum(h3 * ad3_ref[...], axis=1, keepdims=True)
    t = a3s + a3d
    ex_self = jnp.exp(jnp.maximum(t, 0.2 * t))
    H3_ref[...] = h3
    A3_ref[...] = jnp.concatenate(
        [a3s, a3d, ex_self, jnp.zeros_like(h3[:, :13])], axis=1)


def _tc_layer3(o1a, o1b, H1, a1, b1p, o2a, o2b, H2, a2, b2p, W3r, as3, ad3):
    R = 1000
    grid = (N // R,)
    row = lambda i: (i, 0)
    const = lambda i: (0, 0)
    return pl.pallas_call(
        _tc_layer3_body,
        grid=grid,
        in_specs=[
            pl.BlockSpec((R, 64), row), pl.BlockSpec((R, 64), row),
            pl.BlockSpec((R, 64), row), pl.BlockSpec((R, 8), row),
            pl.BlockSpec((1, 64), const),
            pl.BlockSpec((R, 64), row), pl.BlockSpec((R, 64), row),
            pl.BlockSpec((R, 64), row), pl.BlockSpec((R, 8), row),
            pl.BlockSpec((1, 64), const),
            pl.BlockSpec((D, C), const),
            pl.BlockSpec((1, C), const), pl.BlockSpec((1, C), const),
        ],
        out_specs=[pl.BlockSpec((R, C), row), pl.BlockSpec((R, 16), row)],
        out_shape=[jax.ShapeDtypeStruct((N, C), _f32),
                   jax.ShapeDtypeStruct((N, 16), _f32)],
    )(o1a, o1b, H1, a1, b1p, o2a, o2b, H2, a2, b2p, W3r, as3, ad3)


@functools.partial(
    pl.kernel,
    out_type=[
        jax.ShapeDtypeStruct((EPAD,), _f32),
        jax.ShapeDtypeStruct((NC * NPAD,), _f32),
    ],
    mesh=_SC_MESH,
    compiler_params=pltpu.CompilerParams(use_tc_tiling_on_sc=False, needs_layout_passes=False),
    scratch_types=[
        pltpu.VMEM((NPAD,), _f32),
        pltpu.VMEM((NPAD,), _f32),
        pltpu.VMEM((K,), _i32),
        pltpu.VMEM((K,), _i32),
        pltpu.VMEM((K,), _f32),
        pltpu.VMEM_SHARED((NPAD,), _f32),
        pltpu.SemaphoreType.DMA,
    ],
)
def _sc_pass1_3(src1, dst1, a3s_t, a3d_t,
                ex3, s3o,
                a3s_l, a3d_l, idx_s, idx_d, exst, acc3, sem1):
    cid = lax.axis_index("c")
    sid = lax.axis_index("s")
    wid = sid * NC + cid

    pltpu.sync_copy(a3s_t, a3s_l)
    pltpu.sync_copy(a3d_t, a3d_l)

    zero16 = jnp.zeros((16,), _f32)

    @pl.loop(0, K // 16)
    def _zero(g):
        exst[pl.ds(g * 16, 16)] = zero16

    pltpu.sync_copy(exst, acc3.at[pl.ds(sid * RPT, K)])
    pltpu.sync_copy(exst.at[pl.ds(0, RPT - K)], acc3.at[pl.ds(sid * RPT + K, RPT - K)])
    plsc.subcore_barrier()

    @pl.loop(0, NCHUNK)
    def _chunk(j):
        base = wid * PER_W + j * K
        pltpu.sync_copy(src1.at[pl.ds(base, K)], idx_s)
        pltpu.sync_copy(dst1.at[pl.ds(base, K)], idx_d)

        @pl.loop(0, K // 16, unroll=2)
        def _grp(g):
            sv = idx_s[pl.ds(g * 16, 16)]
            dv = idx_d[pl.ds(g * 16, 16)]
            av = plsc.load_gather(a3s_l, [sv])
            bv = plsc.load_gather(a3d_l, [dv])
            v = av + bv
            exst[pl.ds(g * 16, 16)] = jnp.exp(jnp.maximum(v, 0.2 * v))

        pltpu.sync_copy(exst, ex3.at[pl.ds(base, K)])
        pltpu.sync_copy(exst, acc3.at[idx_d], add=True)

    plsc.subcore_barrier()
    rs = pl.ds(sid * RPT, RPT)
    pltpu.sync_copy(acc3.at[rs], s3o.at[pl.ds(cid * NPAD + sid * RPT, RPT)])


def _tc_mid3_body(s3_ref, X3_ref, r3_ref, a3_ref):
    s = s3_ref[...]
    ex_self = X3_ref[...]
    sa = s[: (NPAD // 16)]
    sb = s[(NPAD // 16):]
    r = 1.0 / (sa + sb + ex_self + 1e-16)
    r3_ref[...] = r
    a3_ref[...] = ex_self * r


def _tc_mid3(s3r, X3r):
    out = jax.ShapeDtypeStruct((NPAD // 16, 16), _f32)
    return pl.pallas_call(
        _tc_mid3_body,
        out_shape=[out, out],
    )(s3r, X3r)


@functools.partial(
    pl.kernel,
    out_type=[
        jax.ShapeDtypeStruct((NC * NPAD, 16), _f32),
    ],
    mesh=_SC_MESH,
    compiler_params=pltpu.CompilerParams(use_tc_tiling_on_sc=False, needs_layout_passes=False),
    scratch_types=[
        pltpu.VMEM((NPAD,), _f32),
        pltpu.VMEM((K,), _i32), pltpu.VMEM((K,), _i32),
        pltpu.VMEM((K,), _i32), pltpu.VMEM((K,), _i32),
        pltpu.VMEM((K, 16), _f32), pltpu.VMEM((K, 16), _f32),
        pltpu.VMEM((K,), _f32), pltpu.VMEM((K,), _f32),
        pltpu.VMEM((K, 16), _f32), pltpu.VMEM((K, 16), _f32),
        pltpu.VMEM((K,), _i32), pltpu.VMEM((K,), _i32),
        pltpu.VMEM_SHARED((NPAD, 16), _f32),
        pltpu.SemaphoreType.DMA,
        pltpu.SemaphoreType.DMA,
        pltpu.SemaphoreType.DMA,
        pltpu.SemaphoreType.DMA,
    ],
)
def _sc_pass2_3(src1, dst1, H3, ex3, r3_t,
                o3,
                r3_l, idx_s0, idx_s1, idx_d0, idx_d1, h3rows0, h3rows1,
                exc0, exc1, stage0, stage1, sidx0, sidx1, acc,
                sem0, sem1, sem_s0, sem_s1):
    cid = lax.axis_index("c")
    sid = lax.axis_index("s")
    wid = sid * NC + cid

    idx_s = (idx_s0, idx_s1)
    idx_d = (idx_d0, idx_d1)
    h3rows = (h3rows0, h3rows1)
    exc = (exc0, exc1)
    stage = (stage0, stage1)
    sidx = (sidx0, sidx1)
    sem = (sem0, sem1)
    sem_s = (sem_s0, sem_s1)

    pltpu.sync_copy(r3_t, r3_l)

    zero16 = jnp.zeros((16,), _f32)

    @pl.loop(0, K)
    def _zero(i):
        stage0[i, :] = zero16

    def load_idx(j, b):
        base = wid * PER_W + j * K
        pltpu.sync_copy(src1.at[pl.ds(base, K)], idx_s[b])
        pltpu.sync_copy(dst1.at[pl.ds(base, K)], idx_d[b])

    def fire(j, b):
        base = wid * PER_W + j * K
        pltpu.async_copy(H3.at[idx_s[b]], h3rows[b], sem[b])
        pltpu.async_copy(ex3.at[pl.ds(base, K)], exc[b], sem[b])

    def wait(j, b):
        base = wid * PER_W + j * K
        pltpu.make_async_copy(H3.at[idx_s[b]], h3rows[b], sem[b]).wait()
        pltpu.make_async_copy(ex3.at[pl.ds(base, K)], exc[b], sem[b]).wait()

    pltpu.sync_copy(stage0, acc.at[pl.ds(sid * RPT, K)])
    pltpu.sync_copy(stage0.at[pl.ds(0, RPT - K)], acc.at[pl.ds(sid * RPT + K, RPT - K)])
    load_idx(0, 0)
    fire(0, 0)
    plsc.subcore_barrier()

    @pl.loop(0, NCHUNK // 2)
    def _pair(j2):
        for b in (0, 1):
            j = j2 * 2 + b
            nb = 1 - b
            nxt = j + 1

            @pl.when(nxt < NCHUNK)
            def _prefetch():
                load_idx(nxt, nb)
                fire(nxt, nb)

            wait(j, b)

            @pl.when(j >= 2)
            def _drain():
                pltpu.make_async_copy(stage[b], acc.at[sidx[b]], sem_s[b]).wait()

            h_b, ex_b, st_b, id_b = h3rows[b], exc[b], stage[b], idx_d[b]

            @pl.loop(0, K // 16, unroll=2)
            def _grp(g):
                b16 = pl.ds(g * 16, 16)
                dv = id_b[b16]
                rv = plsc.load_gather(r3_l, [dv])
                alpha = ex_b[b16] * rv
                for e in range(16):
                    row = g * 16 + e
                    st_b[row, :] = h_b[row, :] * alpha[e]

            si_b = sidx[b]

            @pl.loop(0, K // 16)
            def _sicopy(g):
                si_b[pl.ds(g * 16, 16)] = id_b[pl.ds(g * 16, 16)]

            pltpu.async_copy(st_b, acc.at[sidx[b]], sem_s[b], add=True)

    for b in (NCHUNK % 2, (NCHUNK - 1) % 2):
        pltpu.make_async_copy(stage[b], acc.at[sidx[b]], sem_s[b]).wait()

    plsc.subcore_barrier()
    rs = pl.ds(sid * RPT, RPT)
    pltpu.sync_copy(acc.at[rs], o3.at[pl.ds(cid * NPAD + sid * RPT, RPT)])


def _tc_final_body(o3a_ref, o3b_ref, H3_ref, a3_ref, b3_ref, out_ref):
    z = (o3a_ref[...] + o3b_ref[...]
         + H3_ref[...] * a3_ref[...]
         + b3_ref[...])
    m = jnp.max(z, axis=1, keepdims=True)
    zm = z - m
    out_ref[...] = zm - jnp.log(jnp.sum(jnp.exp(zm), axis=1, keepdims=True))


def _tc_final(o3a, o3b, H3, aself3, b3):
    R = 1000
    grid = (N // R,)
    row = lambda i: (i, 0)
    const = lambda i: (0, 0)
    return pl.pallas_call(
        _tc_final_body,
        grid=grid,
        in_specs=[
            pl.BlockSpec((R, C), row), pl.BlockSpec((R, C), row),
            pl.BlockSpec((R, C), row), pl.BlockSpec((R, 1), row),
            pl.BlockSpec((1, C), const),
        ],
        out_specs=pl.BlockSpec((R, C), row),
        out_shape=jax.ShapeDtypeStruct((N, C), _f32),
    )(o3a, o3b, H3, aself3, b3)


def _build_B(att):
    rows = np.arange(64)
    cols = rows % 8
    return jnp.zeros((64, 8), _f32).at[rows, cols].set(att.T.reshape(64))


def _pad_edges(ei):
    npd = EPAD - E
    pad_src = jnp.zeros((npd,), _i32)
    pad_dst = jnp.asarray(N + (np.arange(npd) % PADROWS), _i32)
    src = jnp.concatenate([ei[0], pad_src])
    dst = jnp.concatenate([ei[1], pad_dst])
    return src, dst


def kernel(x, edge_index, topo_edges, W1, as1, ad1, b1, W2, as2, ad2, b2,
           W3, as3, ad3, b3):
    W1p = W1[:, _PERM]
    W2p = W2[:, _PERM]
    B1s, B1d = _build_B(as1), _build_B(ad1)
    B2s, B2d = _build_B(as2), _build_B(ad2)
    b1p = b1[_PERM][None, :]
    b2p = b2[_PERM][None, :]
    orig_perm = np.concatenate([_ORIG, _ORIG + 64])
    W3r = W3[orig_perm, :]

    src1, dst1 = _pad_edges(edge_index)
    src2, dst2 = _pad_edges(topo_edges)

    H1, S1, D1, X1, H2, S2, D2, X2 = _tc_prep12(x, W1p, B1s, B1d, W2p, B2s, B2d)

    zpad16 = jnp.zeros((NPAD - N, 16), _f32)
    zpad8 = jnp.zeros((NPAD - N, 8), _f32)
    npad16 = jnp.full((NPAD - N, 16), -1e30, _f32)
    S1p = jnp.concatenate([S1, zpad16])
    D1p = jnp.concatenate([D1, npad16])
    S2p = jnp.concatenate([S2, zpad16])
    D2p = jnp.concatenate([D2, npad16])
    X1p = jnp.concatenate([X1, zpad8])
    X2p = jnp.concatenate([X2, zpad8])

    ex1, ex2, s1, s2 = _sc_pass1_12(src1, dst1, src2, dst2, S1p, D1p, S2p, D2p)

    r1, a1self, r2, a2self = _tc_mid(s1, X1p, s2, X2p)

    o1, o2 = _sc_pass2_12(src1, dst1, src2, dst2, H1, H2, ex1, ex2, r1, r2)

    H3, A3 = _tc_layer3(o1[:N], o1[NPAD:NPAD + N], H1, a1self[:N], b1p,
                        o2[:N], o2[NPAD:NPAD + N], H2, a2self[:N], b2p,
                        W3r, as3, ad3)

    zpad1 = jnp.zeros((NPAD - N,), _f32)
    a3s_t = jnp.concatenate([A3[:, 0], zpad1])
    a3d_t = jnp.concatenate([A3[:, 1], jnp.full((NPAD - N,), -1e30, _f32)])
    X3p = jnp.concatenate([A3[:, 2], zpad1])

    ex3, s3 = _sc_pass1_3(src1, dst1, a3s_t, a3d_t)

    r3, a3self = _tc_mid3(s3.reshape(2 * NPAD // 16, 16), X3p.reshape(NPAD // 16, 16))

    (o3,) = _sc_pass2_3(src1, dst1, H3, ex3, r3.reshape(-1))

    out = _tc_final(o3[:N], o3[NPAD:NPAD + N], H3,
                    a3self.reshape(-1)[:N, None], b3[None, :])
    return out


# speedup vs baseline: 1.0376x; 1.0376x over previous
import functools

import numpy as np
import jax
import jax.numpy as jnp
from jax import lax
from jax.experimental import pallas as pl
from jax.experimental.pallas import tpu as pltpu
from jax.experimental.pallas import tpu_sc as plsc

N = 10000
E = 320000
D = 128
C = 16

NC = 2
NS = 16
NW = NC * NS
K = 1024
PER_W = 10240
EPAD = NW * PER_W
NCHUNK = PER_W // K
NPAD = 10112
RPT = NPAD // NS
PADROWS = NPAD - N

_f32 = jnp.float32
_i32 = jnp.int32

_SC_MESH = plsc.VectorSubcoreMesh(core_axis_name="c", subcore_axis_name="s")

_PERM = np.array([(p % 8) * 8 + p // 8 for p in range(64)])
_ORIG = np.array([(p % 8) * 8 + p // 8 for p in range(64)])


def _tc_prep12_body(x_ref, W1_ref, B1s_ref, B1d_ref, W2_ref, B2s_ref, B2d_ref,
                    H1_ref, S1_ref, D1_ref, X1_ref, H2_ref, S2_ref, D2_ref, X2_ref):
    xb = x_ref[...]
    for (W_ref, Bs_ref, Bd_ref, H_ref, S_ref, D_ref, X_ref) in (
        (W1_ref, B1s_ref, B1d_ref, H1_ref, S1_ref, D1_ref, X1_ref),
        (W2_ref, B2s_ref, B2d_ref, H2_ref, S2_ref, D2_ref, X2_ref),
    ):
        h = jnp.dot(xb, W_ref[...], preferred_element_type=_f32)
        a_s = jnp.dot(h, Bs_ref[...], preferred_element_type=_f32)
        a_d = jnp.dot(h, Bd_ref[...], preferred_element_type=_f32)
        H_ref[...] = h
        S_ref[...] = jnp.concatenate([a_s, a_s], axis=1)
        D_ref[...] = jnp.concatenate([a_d, a_d], axis=1)
        t = a_s + a_d
        ex = jnp.exp(jnp.maximum(t, 0.2 * t))
        X_ref[...] = jnp.concatenate([ex, ex], axis=1)


def _tc_prep12(x, W1p, B1s, B1d, W2p, B2s, B2d):
    R = 1000
    grid = (N // R,)
    row = lambda i: (i, 0)
    const = lambda i: (0, 0)
    out16 = jax.ShapeDtypeStruct((N, 16), _f32)
    out8 = jax.ShapeDtypeStruct((N, 8), _f32)
    return pl.pallas_call(
        _tc_prep12_body,
        grid=grid,
        in_specs=[
            pl.BlockSpec((R, D), row),
            pl.BlockSpec((D, 64), const), pl.BlockSpec((64, 8), const), pl.BlockSpec((64, 8), const),
            pl.BlockSpec((D, 64), const), pl.BlockSpec((64, 8), const), pl.BlockSpec((64, 8), const),
        ],
        out_specs=[
            pl.BlockSpec((R, 64), row), pl.BlockSpec((R, 16), row),
            pl.BlockSpec((R, 16), row), pl.BlockSpec((R, 16), row),
            pl.BlockSpec((R, 64), row), pl.BlockSpec((R, 16), row),
            pl.BlockSpec((R, 16), row), pl.BlockSpec((R, 16), row),
        ],
        out_shape=[jax.ShapeDtypeStruct((N, 64), _f32), out16, out16, out16,
                   jax.ShapeDtypeStruct((N, 64), _f32), out16, out16, out16],
    )(x, W1p, B1s, B1d, W2p, B2s, B2d)


@functools.partial(
    pl.kernel,
    out_type=[
        jax.ShapeDtypeStruct((EPAD, 16), _f32),
        jax.ShapeDtypeStruct((EPAD, 16), _f32),
        jax.ShapeDtypeStruct((NC * NPAD, 16), _f32),
        jax.ShapeDtypeStruct((NC * NPAD, 16), _f32),
    ],
    mesh=_SC_MESH,
    compiler_params=pltpu.CompilerParams(use_tc_tiling_on_sc=False, needs_layout_passes=False),
    scratch_types=[
        pltpu.VMEM((K,), _i32), pltpu.VMEM((K,), _i32),
        pltpu.VMEM((K,), _i32), pltpu.VMEM((K,), _i32),
        pltpu.VMEM((K, 16), _f32), pltpu.VMEM((K, 16), _f32),
        pltpu.VMEM((K, 16), _f32), pltpu.VMEM((K, 16), _f32),
        pltpu.VMEM((K, 16), _f32), pltpu.VMEM((K, 16), _f32),
        pltpu.VMEM((K,), _i32), pltpu.VMEM((K,), _i32),
        pltpu.VMEM_SHARED((NPAD, 16), _f32),
        pltpu.VMEM_SHARED((NPAD, 16), _f32),
        pltpu.SemaphoreType.DMA,
        pltpu.SemaphoreType.DMA,
        pltpu.SemaphoreType.DMA,
        pltpu.SemaphoreType.DMA,
        pltpu.SemaphoreType.DMA,
        pltpu.SemaphoreType.DMA,
    ],
)
def _sc_pass1_12(src1, dst1, src2, dst2, S1, D1, S2, D2,
                 ex1, ex2, s1o, s2o,
                 idx_s0, idx_s1, idx_d0, idx_d1, rows_s0, rows_s1,
                 rows_d0, rows_d1, stage0, stage1,
                 sidx0, sidx1, acc1, acc2,
                 sem0, sem1, sem_s0, sem_s1, sem_e0, sem_e1):
    cid = lax.axis_index("c")
    sid = lax.axis_index("s")
    wid = sid * NC + cid

    idx_s = (idx_s0, idx_s1)
    idx_d = (idx_d0, idx_d1)
    rows_s = (rows_s0, rows_s1)
    rows_d = (rows_d0, rows_d1)
    stage = (stage0, stage1)
    sidx = (sidx0, sidx1)
    sem = (sem0, sem1)
    sem_s = (sem_s0, sem_s1)
    sem_e = (sem_e0, sem_e1)

    zero16 = jnp.zeros((16,), _f32)

    @pl.loop(0, K)
    def _zero(i):
        stage0[i, :] = zero16

    for acc in (acc1, acc2):
        pltpu.sync_copy(stage0.at[pl.ds(0, RPT)], acc.at[pl.ds(sid * RPT, RPT)])
    plsc.subcore_barrier()

    for (src, dst, S, Dt, exo, acc) in ((src1, dst1, S1, D1, ex1, acc1),
                                        (src2, dst2, S2, D2, ex2, acc2)):
        def load_idx(j, b):
            base = wid * PER_W + j * K
            pltpu.sync_copy(src.at[pl.ds(base, K)], idx_s[b])
            pltpu.sync_copy(dst.at[pl.ds(base, K)], idx_d[b])

        def fire(b):
            pltpu.async_copy(S.at[idx_s[b]], rows_s[b], sem[b])
            pltpu.async_copy(Dt.at[idx_d[b]], rows_d[b], sem[b])

        def wait(b):
            pltpu.make_async_copy(S.at[idx_s[b]], rows_s[b], sem[b]).wait()
            pltpu.make_async_copy(Dt.at[idx_d[b]], rows_d[b], sem[b]).wait()

        load_idx(0, 0)
        fire(0)

        @pl.loop(0, NCHUNK // 2)
        def _pair(j2):
            for b in (0, 1):
                j = j2 * 2 + b
                nb = 1 - b
                nxt = j + 1

                @pl.when(nxt < NCHUNK)
                def _prefetch():
                    load_idx(nxt, nb)
                    fire(nb)

                wait(b)

                @pl.when(j >= 2)
                def _drain():
                    base_p = wid * PER_W + (j - 2) * K
                    pltpu.make_async_copy(stage[b], exo.at[pl.ds(base_p, K)], sem_e[b]).wait()
                    pltpu.make_async_copy(stage[b], acc.at[sidx[b]], sem_s[b]).wait()

                rs_b, rd_b, st_b = rows_s[b], rows_d[b], stage[b]

                @pl.loop(0, K, unroll=4)
                def _edge(i):
                    v = rs_b[i, :] + rd_b[i, :]
                    st_b[i, :] = jnp.exp(jnp.maximum(v, 0.2 * v))

                base = wid * PER_W + j * K
                id_b, si_b = idx_d[b], sidx[b]

                @pl.loop(0, K // 16)
                def _sicopy(g):
                    si_b[pl.ds(g * 16, 16)] = id_b[pl.ds(g * 16, 16)]

                pltpu.async_copy(st_b, exo.at[pl.ds(base, K)], sem_e[b])
                pltpu.async_copy(st_b, acc.at[sidx[b]], sem_s[b], add=True)

        for b, j_last in ((NCHUNK % 2, NCHUNK - 2), ((NCHUNK - 1) % 2, NCHUNK - 1)):
            base_p = wid * PER_W + j_last * K
            pltpu.make_async_copy(stage[b], exo.at[pl.ds(base_p, K)], sem_e[b]).wait()
            pltpu.make_async_copy(stage[b], acc.at[sidx[b]], sem_s[b]).wait()

    plsc.subcore_barrier()
    rs = pl.ds(sid * RPT, RPT)
    pltpu.sync_copy(acc1.at[rs], s1o.at[pl.ds(cid * NPAD + sid * RPT, RPT)])
    pltpu.sync_copy(acc2.at[rs], s2o.at[pl.ds(cid * NPAD + sid * RPT, RPT)])


def _tc_mid_body(s1_ref, X1_ref, s2_ref, X2_ref, r1_ref, a1_ref, r2_ref, a2_ref):
    for (s_ref, X_ref, r_ref, a_ref) in ((s1_ref, X1_ref, r1_ref, a1_ref),
                                         (s2_ref, X2_ref, r2_ref, a2_ref)):
        s = s_ref[...]
        ex_self = X_ref[...]
        tot = s[:NPAD] + s[NPAD:] + ex_self
        r = 1.0 / (tot + 1e-16)
        r_ref[...] = r
        a_ref[...] = ex_self * r


def _tc_mid(s1, X1p, s2, X2p):
    out = jax.ShapeDtypeStruct((NPAD, 16), _f32)
    return pl.pallas_call(
        _tc_mid_body,
        out_shape=[out, out, out, out],
    )(s1, X1p, s2, X2p)


K2 = 256
NCHUNK2 = PER_W // K2


@functools.partial(
    pl.kernel,
    out_type=[
        jax.ShapeDtypeStruct((NC * NPAD, 64), _f32),
        jax.ShapeDtypeStruct((NC * NPAD, 64), _f32),
    ],
    mesh=_SC_MESH,
    compiler_params=pltpu.CompilerParams(use_tc_tiling_on_sc=False, needs_layout_passes=False),
    scratch_types=[
        pltpu.VMEM((K2,), _i32), pltpu.VMEM((K2,), _i32),
        pltpu.VMEM((K2,), _i32), pltpu.VMEM((K2,), _i32),
        pltpu.VMEM((K2, 64), _f32), pltpu.VMEM((K2, 64), _f32),
        pltpu.VMEM((K2, 16), _f32), pltpu.VMEM((K2, 16), _f32),
        pltpu.VMEM((K2, 16), _f32), pltpu.VMEM((K2, 16), _f32),
        pltpu.VMEM((K2, 64), _f32), pltpu.VMEM((K2, 64), _f32),
        pltpu.VMEM((K2,), _i32), pltpu.VMEM((K2,), _i32),
        pltpu.VMEM_SHARED((NPAD, 64), _f32),
        pltpu.SemaphoreType.DMA,
        pltpu.SemaphoreType.DMA,
        pltpu.SemaphoreType.DMA,
        pltpu.SemaphoreType.DMA,
    ],
)
def _sc_pass2_12(src1, dst1, src2, dst2, H1, H2, ex1, ex2, r1, r2,
                 o1, o2,
                 idx_s0, idx_s1, idx_d0, idx_d1, hrows0, hrows1,
                 exrows0, exrows1, rrows0, rrows1, stage0, stage1,
                 sidx0, sidx1, acc,
                 sem0, sem1, sem_s0, sem_s1):
    cid = lax.axis_index("c")
    sid = lax.axis_index("s")
    wid = sid * NC + cid

    idx_s = (idx_s0, idx_s1)
    idx_d = (idx_d0, idx_d1)
    hrows = (hrows0, hrows1)
    exrows = (exrows0, exrows1)
    rrows = (rrows0, rrows1)
    stage = (stage0, stage1)
    sidx = (sidx0, sidx1)
    sem = (sem0, sem1)
    sem_s = (sem_s0, sem_s1)

    zero16 = jnp.zeros((16,), _f32)
    rs = pl.ds(sid * RPT, RPT)

    for (src, dst, H, exi, r, oo) in ((src1, dst1, H1, ex1, r1, o1),
                                      (src2, dst2, H2, ex2, r2, o2)):
        def load_idx(j, b):
            base = wid * PER_W + j * K2
            pltpu.sync_copy(src.at[pl.ds(base, K2)], idx_s[b])
            pltpu.sync_copy(dst.at[pl.ds(base, K2)], idx_d[b])

        def fire(j, b):
            base = wid * PER_W + j * K2
            pltpu.async_copy(H.at[idx_s[b]], hrows[b], sem[b])
            pltpu.async_copy(r.at[idx_d[b]], rrows[b], sem[b])
            pltpu.async_copy(exi.at[pl.ds(base, K2)], exrows[b], sem[b])

        def wait(j, b):
            base = wid * PER_W + j * K2
            pltpu.make_async_copy(H.at[idx_s[b]], hrows[b], sem[b]).wait()
            pltpu.make_async_copy(r.at[idx_d[b]], rrows[b], sem[b]).wait()
            pltpu.make_async_copy(exi.at[pl.ds(base, K2)], exrows[b], sem[b]).wait()

        @pl.loop(0, K2)
        def _zero(i):
            for q in range(4):
                stage0[i, pl.ds(q * 16, 16)] = zero16

        off = 0
        while off < RPT:
            n = min(K2, RPT - off)
            pltpu.sync_copy(stage0.at[pl.ds(0, n)],
                            acc.at[pl.ds(sid * RPT + off, n)])
            off += n
        load_idx(0, 0)
        fire(0, 0)
        plsc.subcore_barrier()

        @pl.loop(0, NCHUNK2 // 2)
        def _pair(j2):
            for b in (0, 1):
                j = j2 * 2 + b
                nb = 1 - b
                nxt = j + 1

                @pl.when(nxt < NCHUNK2)
                def _prefetch():
                    load_idx(nxt, nb)
                    fire(nxt, nb)

                wait(j, b)

                @pl.when(j >= 2)
                def _drain():
                    pltpu.make_async_copy(stage[b], acc.at[sidx[b]], sem_s[b]).wait()

                h_b, ex_b, r_b, st_b = hrows[b], exrows[b], rrows[b], stage[b]

                @pl.loop(0, K2, unroll=2)
                def _edge(i):
                    alpha = ex_b[i, :] * r_b[i, :]
                    for q in range(4):
                        st_b[i, pl.ds(q * 16, 16)] = h_b[i, pl.ds(q * 16, 16)] * alpha

                id_b, si_b = idx_d[b], sidx[b]

                @pl.loop(0, K2 // 16)
                def _sicopy(g):
                    si_b[pl.ds(g * 16, 16)] = id_b[pl.ds(g * 16, 16)]

                pltpu.async_copy(st_b, acc.at[sidx[b]], sem_s[b], add=True)

        for b in (NCHUNK2 % 2, (NCHUNK2 - 1) % 2):
            pltpu.make_async_copy(stage[b], acc.at[sidx[b]], sem_s[b]).wait()

        plsc.subcore_barrier()
        pltpu.sync_copy(acc.at[rs], oo.at[pl.ds(cid * NPAD + sid * RPT, RPT)])
        plsc.subcore_barrier()


def _tc_layer3_body(o1a_ref, o1b_ref, H1_ref, a1_ref, b1_ref,
                    o2a_ref, o2b_ref, H2_ref, a2_ref, b2_ref,
                    W3_ref, as3_ref, ad3_ref,
                    H3_ref, A3_ref):
    xs = []
    for (oa, ob, H_ref, a_ref, b_ref) in ((o1a_ref, o1b_ref, H1_ref, a1_ref, b1_ref),
                                          (o2a_ref, o2b_ref, H2_ref, a2_ref, b2_ref)):
        aself = a_ref[...][:, :8]
        xl = oa[...] + ob[...] + H_ref[...] * jnp.tile(aself, (1, 8)) + b_ref[...]
        xl = jnp.where(xl > 0, xl, jnp.exp(jnp.minimum(xl, 0.0)) - 1.0)
        xs.append(xl)
    xc = jnp.concatenate(xs, axis=1)
    h3 = jnp.dot(xc, W3_ref[...], preferred_element_type=_f32)
    a3s = jnp.sum(h3 * as3_ref[...], axis=1, keepdims=True)
    a3d = jnp.sum(h3 * ad3_ref[...], axis=1, keepdims=True)
    t = a3s + a3d
    ex_self = jnp.exp(jnp.maximum(t, 0.2 * t))
    H3_ref[...] = h3
    A3_ref[...] = jnp.concatenate(
        [a3s, a3d, ex_self, jnp.zeros_like(h3[:, :13])], axis=1)


def _tc_layer3(o1a, o1b, H1, a1, b1p, o2a, o2b, H2, a2, b2p, W3r, as3, ad3):
    R = 1000
    grid = (N // R,)
    row = lambda i: (i, 0)
    const = lambda i: (0, 0)
    return pl.pallas_call(
        _tc_layer3_body,
        grid=grid,
        in_specs=[
            pl.BlockSpec((R, 64), row), pl.BlockSpec((R, 64), row),
            pl.BlockSpec((R, 64), row), pl.BlockSpec((R, 16), row),
            pl.BlockSpec((1, 64), const),
            pl.BlockSpec((R, 64), row), pl.BlockSpec((R, 64), row),
            pl.BlockSpec((R, 64), row), pl.BlockSpec((R, 16), row),
            pl.BlockSpec((1, 64), const),
            pl.BlockSpec((D, C), const),
            pl.BlockSpec((1, C), const), pl.BlockSpec((1, C), const),
        ],
        out_specs=[pl.BlockSpec((R, C), row), pl.BlockSpec((R, 16), row)],
        out_shape=[jax.ShapeDtypeStruct((N, C), _f32),
                   jax.ShapeDtypeStruct((N, 16), _f32)],
    )(o1a, o1b, H1, a1, b1p, o2a, o2b, H2, a2, b2p, W3r, as3, ad3)


@functools.partial(
    pl.kernel,
    out_type=[
        jax.ShapeDtypeStruct((EPAD,), _f32),
        jax.ShapeDtypeStruct((NC * NPAD,), _f32),
    ],
    mesh=_SC_MESH,
    compiler_params=pltpu.CompilerParams(use_tc_tiling_on_sc=False, needs_layout_passes=False),
    scratch_types=[
        pltpu.VMEM((NPAD,), _f32),
        pltpu.VMEM((NPAD,), _f32),
        pltpu.VMEM((K,), _i32),
        pltpu.VMEM((K,), _i32),
        pltpu.VMEM((K,), _f32),
        pltpu.VMEM_SHARED((NPAD,), _f32),
        pltpu.SemaphoreType.DMA,
    ],
)
def _sc_pass1_3(src1, dst1, a3s_t, a3d_t,
                ex3, s3o,
                a3s_l, a3d_l, idx_s, idx_d, exst, acc3, sem1):
    cid = lax.axis_index("c")
    sid = lax.axis_index("s")
    wid = sid * NC + cid

    pltpu.sync_copy(a3s_t, a3s_l)
    pltpu.sync_copy(a3d_t, a3d_l)

    zero16 = jnp.zeros((16,), _f32)

    @pl.loop(0, K // 16)
    def _zero(g):
        exst[pl.ds(g * 16, 16)] = zero16

    pltpu.sync_copy(exst.at[pl.ds(0, RPT)], acc3.at[pl.ds(sid * RPT, RPT)])
    plsc.subcore_barrier()

    @pl.loop(0, NCHUNK)
    def _chunk(j):
        base = wid * PER_W + j * K
        pltpu.sync_copy(src1.at[pl.ds(base, K)], idx_s)
        pltpu.sync_copy(dst1.at[pl.ds(base, K)], idx_d)

        @pl.loop(0, K // 16, unroll=2)
        def _grp(g):
            sv = idx_s[pl.ds(g * 16, 16)]
            dv = idx_d[pl.ds(g * 16, 16)]
            av = plsc.load_gather(a3s_l, [sv])
            bv = plsc.load_gather(a3d_l, [dv])
            v = av + bv
            exst[pl.ds(g * 16, 16)] = jnp.exp(jnp.maximum(v, 0.2 * v))

        pltpu.sync_copy(exst, ex3.at[pl.ds(base, K)])
        pltpu.sync_copy(exst, acc3.at[idx_d], add=True)

    plsc.subcore_barrier()
    rs = pl.ds(sid * RPT, RPT)
    pltpu.sync_copy(acc3.at[rs], s3o.at[pl.ds(cid * NPAD + sid * RPT, RPT)])


def _tc_mid3_body(s3_ref, X3_ref, r3_ref, a3_ref):
    s = s3_ref[...]
    ex_self = X3_ref[...]
    sa = s[: (NPAD // 16)]
    sb = s[(NPAD // 16):]
    r = 1.0 / (sa + sb + ex_self + 1e-16)
    r3_ref[...] = r
    a3_ref[...] = ex_self * r


def _tc_mid3(s3r, X3r):
    out = jax.ShapeDtypeStruct((NPAD // 16, 16), _f32)
    return pl.pallas_call(
        _tc_mid3_body,
        out_shape=[out, out],
    )(s3r, X3r)


@functools.partial(
    pl.kernel,
    out_type=[
        jax.ShapeDtypeStruct((NC * NPAD, 16), _f32),
    ],
    mesh=_SC_MESH,
    compiler_params=pltpu.CompilerParams(use_tc_tiling_on_sc=False, needs_layout_passes=False),
    scratch_types=[
        pltpu.VMEM((NPAD,), _f32),
        pltpu.VMEM((K,), _i32), pltpu.VMEM((K,), _i32),
        pltpu.VMEM((K,), _i32), pltpu.VMEM((K,), _i32),
        pltpu.VMEM((K, 16), _f32), pltpu.VMEM((K, 16), _f32),
        pltpu.VMEM((K,), _f32), pltpu.VMEM((K,), _f32),
        pltpu.VMEM((K, 16), _f32), pltpu.VMEM((K, 16), _f32),
        pltpu.VMEM((K,), _i32), pltpu.VMEM((K,), _i32),
        pltpu.VMEM_SHARED((NPAD, 16), _f32),
        pltpu.SemaphoreType.DMA,
        pltpu.SemaphoreType.DMA,
        pltpu.SemaphoreType.DMA,
        pltpu.SemaphoreType.DMA,
    ],
)
def _sc_pass2_3(src1, dst1, H3, ex3, r3_t,
                o3,
                r3_l, idx_s0, idx_s1, idx_d0, idx_d1, h3rows0, h3rows1,
                exc0, exc1, stage0, stage1, sidx0, sidx1, acc,
                sem0, sem1, sem_s0, sem_s1):
    cid = lax.axis_index("c")
    sid = lax.axis_index("s")
    wid = sid * NC + cid

    idx_s = (idx_s0, idx_s1)
    idx_d = (idx_d0, idx_d1)
    h3rows = (h3rows0, h3rows1)
    exc = (exc0, exc1)
    stage = (stage0, stage1)
    sidx = (sidx0, sidx1)
    sem = (sem0, sem1)
    sem_s = (sem_s0, sem_s1)

    pltpu.sync_copy(r3_t, r3_l)

    zero16 = jnp.zeros((16,), _f32)

    @pl.loop(0, K)
    def _zero(i):
        stage0[i, :] = zero16

    def load_idx(j, b):
        base = wid * PER_W + j * K
        pltpu.sync_copy(src1.at[pl.ds(base, K)], idx_s[b])
        pltpu.sync_copy(dst1.at[pl.ds(base, K)], idx_d[b])

    def fire(j, b):
        base = wid * PER_W + j * K
        pltpu.async_copy(H3.at[idx_s[b]], h3rows[b], sem[b])
        pltpu.async_copy(ex3.at[pl.ds(base, K)], exc[b], sem[b])

    def wait(j, b):
        base = wid * PER_W + j * K
        pltpu.make_async_copy(H3.at[idx_s[b]], h3rows[b], sem[b]).wait()
        pltpu.make_async_copy(ex3.at[pl.ds(base, K)], exc[b], sem[b]).wait()

    pltpu.sync_copy(stage0.at[pl.ds(0, RPT)], acc.at[pl.ds(sid * RPT, RPT)])
    load_idx(0, 0)
    fire(0, 0)
    plsc.subcore_barrier()

    @pl.loop(0, NCHUNK // 2)
    def _pair(j2):
        for b in (0, 1):
            j = j2 * 2 + b
            nb = 1 - b
            nxt = j + 1

            @pl.when(nxt < NCHUNK)
            def _prefetch():
                load_idx(nxt, nb)
                fire(nxt, nb)

            wait(j, b)

            @pl.when(j >= 2)
            def _drain():
                pltpu.make_async_copy(stage[b], acc.at[sidx[b]], sem_s[b]).wait()

            h_b, ex_b, st_b, id_b = h3rows[b], exc[b], stage[b], idx_d[b]

            @pl.loop(0, K // 16, unroll=2)
            def _grp(g):
                b16 = pl.ds(g * 16, 16)
                dv = id_b[b16]
                rv = plsc.load_gather(r3_l, [dv])
                alpha = ex_b[b16] * rv
                for e in range(16):
                    row = g * 16 + e
                    st_b[row, :] = h_b[row, :] * alpha[e]

            si_b = sidx[b]

            @pl.loop(0, K // 16)
            def _sicopy(g):
                si_b[pl.ds(g * 16, 16)] = id_b[pl.ds(g * 16, 16)]

            pltpu.async_copy(st_b, acc.at[sidx[b]], sem_s[b], add=True)

    for b in (NCHUNK % 2, (NCHUNK - 1) % 2):
        pltpu.make_async_copy(stage[b], acc.at[sidx[b]], sem_s[b]).wait()

    plsc.subcore_barrier()
    rs = pl.ds(sid * RPT, RPT)
    pltpu.sync_copy(acc.at[rs], o3.at[pl.ds(cid * NPAD + sid * RPT, RPT)])


def _tc_final_body(o3a_ref, o3b_ref, H3_ref, a3_ref, b3_ref, out_ref):
    z = (o3a_ref[...] + o3b_ref[...]
         + H3_ref[...] * a3_ref[...]
         + b3_ref[...])
    m = jnp.max(z, axis=1, keepdims=True)
    zm = z - m
    out_ref[...] = zm - jnp.log(jnp.sum(jnp.exp(zm), axis=1, keepdims=True))


def _tc_final(o3a, o3b, H3, aself3, b3):
    R = 1000
    grid = (N // R,)
    row = lambda i: (i, 0)
    const = lambda i: (0, 0)
    return pl.pallas_call(
        _tc_final_body,
        grid=grid,
        in_specs=[
            pl.BlockSpec((R, C), row), pl.BlockSpec((R, C), row),
            pl.BlockSpec((R, C), row), pl.BlockSpec((R, 1), row),
            pl.BlockSpec((1, C), const),
        ],
        out_specs=pl.BlockSpec((R, C), row),
        out_shape=jax.ShapeDtypeStruct((N, C), _f32),
    )(o3a, o3b, H3, aself3, b3)


def _build_B(att):
    rows = np.arange(64)
    cols = rows % 8
    return jnp.zeros((64, 8), _f32).at[rows, cols].set(att.T.reshape(64))


def _pad_edges(ei):
    npd = EPAD - E
    pad_src = jnp.zeros((npd,), _i32)
    pad_dst = jnp.asarray(N + (np.arange(npd) % PADROWS), _i32)
    src = jnp.concatenate([ei[0], pad_src])
    dst = jnp.concatenate([ei[1], pad_dst])
    return src, dst


def kernel(x, edge_index, topo_edges, W1, as1, ad1, b1, W2, as2, ad2, b2,
           W3, as3, ad3, b3):
    W1p = W1[:, _PERM]
    W2p = W2[:, _PERM]
    B1s, B1d = _build_B(as1), _build_B(ad1)
    B2s, B2d = _build_B(as2), _build_B(ad2)
    b1p = b1[_PERM][None, :]
    b2p = b2[_PERM][None, :]
    orig_perm = np.concatenate([_ORIG, _ORIG + 64])
    W3r = W3[orig_perm, :]

    src1, dst1 = _pad_edges(edge_index)
    src2, dst2 = _pad_edges(topo_edges)

    H1, S1, D1, X1, H2, S2, D2, X2 = _tc_prep12(x, W1p, B1s, B1d, W2p, B2s, B2d)

    zpad16 = jnp.zeros((NPAD - N, 16), _f32)
    npad16 = jnp.full((NPAD - N, 16), -1e30, _f32)
    S1p = jnp.concatenate([S1, zpad16])
    D1p = jnp.concatenate([D1, npad16])
    S2p = jnp.concatenate([S2, zpad16])
    D2p = jnp.concatenate([D2, npad16])
    X1p = jnp.concatenate([X1, zpad16])
    X2p = jnp.concatenate([X2, zpad16])

    ex1, ex2, s1, s2 = _sc_pass1_12(src1, dst1, src2, dst2, S1p, D1p, S2p, D2p)

    r1, a1self, r2, a2self = _tc_mid(s1, X1p, s2, X2p)

    o1, o2 = _sc_pass2_12(src1, dst1, src2, dst2, H1, H2, ex1, ex2, r1, r2)

    H3, A3 = _tc_layer3(o1[:N], o1[NPAD:NPAD + N], H1, a1self[:N], b1p,
                        o2[:N], o2[NPAD:NPAD + N], H2, a2self[:N], b2p,
                        W3r, as3, ad3)

    zpad1 = jnp.zeros((NPAD - N,), _f32)
    a3s_t = jnp.concatenate([A3[:, 0], zpad1])
    a3d_t = jnp.concatenate([A3[:, 1], jnp.full((NPAD - N,), -1e30, _f32)])
    X3p = jnp.concatenate([A3[:, 2], zpad1])

    ex3, s3 = _sc_pass1_3(src1, dst1, a3s_t, a3d_t)

    r3, a3self = _tc_mid3(s3.reshape(2 * NPAD // 16, 16), X3p.reshape(NPAD // 16, 16))

    (o3,) = _sc_pass2_3(src1, dst1, H3, ex3, r3.reshape(-1))

    out = _tc_final(o3[:N], o3[NPAD:NPAD + N], H3,
                    a3self.reshape(-1)[:N, None], b3[None, :])
    return out


# speedup vs baseline: 1.0416x; 1.0038x over previous
import functools

import numpy as np
import jax
import jax.numpy as jnp
from jax import lax
from jax.experimental import pallas as pl
from jax.experimental.pallas import tpu as pltpu
from jax.experimental.pallas import tpu_sc as plsc

N = 10000
E = 320000
D = 128
C = 16

NC = 2
NS = 16
NW = NC * NS
K = 1024
PER_W = 10240
EPAD = NW * PER_W
NCHUNK = PER_W // K
NPAD = 10112
RPT = NPAD // NS
PADROWS = NPAD - N

_f32 = jnp.float32
_i32 = jnp.int32

_SC_MESH = plsc.VectorSubcoreMesh(core_axis_name="c", subcore_axis_name="s")

_PERM = np.array([(p % 8) * 8 + p // 8 for p in range(64)])
_ORIG = np.array([(p % 8) * 8 + p // 8 for p in range(64)])


def _tc_prep12_body(x_ref, W1_ref, B1s_ref, B1d_ref, W2_ref, B2s_ref, B2d_ref,
                    H1_ref, S1_ref, D1_ref, X1_ref, H2_ref, S2_ref, D2_ref, X2_ref):
    xb = x_ref[...]
    for (W_ref, Bs_ref, Bd_ref, H_ref, S_ref, D_ref, X_ref) in (
        (W1_ref, B1s_ref, B1d_ref, H1_ref, S1_ref, D1_ref, X1_ref),
        (W2_ref, B2s_ref, B2d_ref, H2_ref, S2_ref, D2_ref, X2_ref),
    ):
        h = jnp.dot(xb, W_ref[...], preferred_element_type=_f32)
        a_s = jnp.dot(h, Bs_ref[...], preferred_element_type=_f32)
        a_d = jnp.dot(h, Bd_ref[...], preferred_element_type=_f32)
        H_ref[...] = h
        S_ref[...] = jnp.concatenate([a_s, a_s], axis=1)
        D_ref[...] = jnp.concatenate([a_d, a_d], axis=1)
        t = a_s + a_d
        ex = jnp.exp(jnp.maximum(t, 0.2 * t))
        X_ref[...] = jnp.concatenate([ex, ex], axis=1)


def _tc_prep12(x, W1p, B1s, B1d, W2p, B2s, B2d):
    R = 1000
    grid = (N // R,)
    row = lambda i: (i, 0)
    const = lambda i: (0, 0)
    out16 = jax.ShapeDtypeStruct((N, 16), _f32)
    out8 = jax.ShapeDtypeStruct((N, 8), _f32)
    return pl.pallas_call(
        _tc_prep12_body,
        grid=grid,
        in_specs=[
            pl.BlockSpec((R, D), row),
            pl.BlockSpec((D, 64), const), pl.BlockSpec((64, 8), const), pl.BlockSpec((64, 8), const),
            pl.BlockSpec((D, 64), const), pl.BlockSpec((64, 8), const), pl.BlockSpec((64, 8), const),
        ],
        out_specs=[
            pl.BlockSpec((R, 64), row), pl.BlockSpec((R, 16), row),
            pl.BlockSpec((R, 16), row), pl.BlockSpec((R, 16), row),
            pl.BlockSpec((R, 64), row), pl.BlockSpec((R, 16), row),
            pl.BlockSpec((R, 16), row), pl.BlockSpec((R, 16), row),
        ],
        out_shape=[jax.ShapeDtypeStruct((N, 64), _f32), out16, out16, out16,
                   jax.ShapeDtypeStruct((N, 64), _f32), out16, out16, out16],
    )(x, W1p, B1s, B1d, W2p, B2s, B2d)


@functools.partial(
    pl.kernel,
    out_type=[
        jax.ShapeDtypeStruct((EPAD, 16), _f32),
        jax.ShapeDtypeStruct((EPAD, 16), _f32),
        jax.ShapeDtypeStruct((NC * NPAD, 16), _f32),
        jax.ShapeDtypeStruct((NC * NPAD, 16), _f32),
    ],
    mesh=_SC_MESH,
    compiler_params=pltpu.CompilerParams(use_tc_tiling_on_sc=False, needs_layout_passes=False),
    scratch_types=[
        pltpu.VMEM((K,), _i32), pltpu.VMEM((K,), _i32),
        pltpu.VMEM((K,), _i32), pltpu.VMEM((K,), _i32),
        pltpu.VMEM((K, 16), _f32), pltpu.VMEM((K, 16), _f32),
        pltpu.VMEM((K, 16), _f32), pltpu.VMEM((K, 16), _f32),
        pltpu.VMEM((K, 16), _f32), pltpu.VMEM((K, 16), _f32),
        pltpu.VMEM((K,), _i32), pltpu.VMEM((K,), _i32),
        pltpu.VMEM_SHARED((NPAD, 16), _f32),
        pltpu.VMEM_SHARED((NPAD, 16), _f32),
        pltpu.SemaphoreType.DMA,
        pltpu.SemaphoreType.DMA,
        pltpu.SemaphoreType.DMA,
        pltpu.SemaphoreType.DMA,
        pltpu.SemaphoreType.DMA,
        pltpu.SemaphoreType.DMA,
    ],
)
def _sc_pass1_12(src1, dst1, src2, dst2, S1, D1, S2, D2,
                 ex1, ex2, s1o, s2o,
                 idx_s0, idx_s1, idx_d0, idx_d1, rows_s0, rows_s1,
                 rows_d0, rows_d1, stage0, stage1,
                 sidx0, sidx1, acc1, acc2,
                 sem0, sem1, sem_s0, sem_s1, sem_e0, sem_e1):
    cid = lax.axis_index("c")
    sid = lax.axis_index("s")
    wid = sid * NC + cid

    idx_s = (idx_s0, idx_s1)
    idx_d = (idx_d0, idx_d1)
    rows_s = (rows_s0, rows_s1)
    rows_d = (rows_d0, rows_d1)
    stage = (stage0, stage1)
    sidx = (sidx0, sidx1)
    sem = (sem0, sem1)
    sem_s = (sem_s0, sem_s1)
    sem_e = (sem_e0, sem_e1)

    zero16 = jnp.zeros((16,), _f32)

    @pl.loop(0, K)
    def _zero(i):
        stage0[i, :] = zero16

    for acc in (acc1, acc2):
        pltpu.sync_copy(stage0.at[pl.ds(0, RPT)], acc.at[pl.ds(sid * RPT, RPT)])
    plsc.subcore_barrier()

    for (src, dst, S, Dt, exo, acc) in ((src1, dst1, S1, D1, ex1, acc1),
                                        (src2, dst2, S2, D2, ex2, acc2)):
        def load_idx(j, b):
            base = wid * PER_W + j * K
            pltpu.sync_copy(src.at[pl.ds(base, K)], idx_s[b])
            pltpu.sync_copy(dst.at[pl.ds(base, K)], idx_d[b])

        def fire(b):
            pltpu.async_copy(S.at[idx_s[b]], rows_s[b], sem[b])
            pltpu.async_copy(Dt.at[idx_d[b]], rows_d[b], sem[b])

        def wait(b):
            pltpu.make_async_copy(S.at[idx_s[b]], rows_s[b], sem[b]).wait()
            pltpu.make_async_copy(Dt.at[idx_d[b]], rows_d[b], sem[b]).wait()

        load_idx(0, 0)
        fire(0)

        @pl.loop(0, NCHUNK // 2)
        def _pair(j2):
            for b in (0, 1):
                j = j2 * 2 + b
                nb = 1 - b
                nxt = j + 1

                @pl.when(nxt < NCHUNK)
                def _prefetch():
                    load_idx(nxt, nb)
                    fire(nb)

                wait(b)

                @pl.when(j >= 2)
                def _drain():
                    base_p = wid * PER_W + (j - 2) * K
                    pltpu.make_async_copy(stage[b], exo.at[pl.ds(base_p, K)], sem_e[b]).wait()
                    pltpu.make_async_copy(stage[b], acc.at[sidx[b]], sem_s[b]).wait()

                rs_b, rd_b, st_b = rows_s[b], rows_d[b], stage[b]

                @pl.loop(0, K, unroll=8)
                def _edge(i):
                    v = rs_b[i, :] + rd_b[i, :]
                    st_b[i, :] = jnp.exp(jnp.maximum(v, 0.2 * v))

                base = wid * PER_W + j * K
                id_b, si_b = idx_d[b], sidx[b]

                @pl.loop(0, K // 16)
                def _sicopy(g):
                    si_b[pl.ds(g * 16, 16)] = id_b[pl.ds(g * 16, 16)]

                pltpu.async_copy(st_b, exo.at[pl.ds(base, K)], sem_e[b])
                pltpu.async_copy(st_b, acc.at[sidx[b]], sem_s[b], add=True)

        for b, j_last in ((NCHUNK % 2, NCHUNK - 2), ((NCHUNK - 1) % 2, NCHUNK - 1)):
            base_p = wid * PER_W + j_last * K
            pltpu.make_async_copy(stage[b], exo.at[pl.ds(base_p, K)], sem_e[b]).wait()
            pltpu.make_async_copy(stage[b], acc.at[sidx[b]], sem_s[b]).wait()

    plsc.subcore_barrier()
    rs = pl.ds(sid * RPT, RPT)
    pltpu.sync_copy(acc1.at[rs], s1o.at[pl.ds(cid * NPAD + sid * RPT, RPT)])
    pltpu.sync_copy(acc2.at[rs], s2o.at[pl.ds(cid * NPAD + sid * RPT, RPT)])


def _tc_mid_body(s1_ref, X1_ref, s2_ref, X2_ref, r1_ref, a1_ref, r2_ref, a2_ref):
    for (s_ref, X_ref, r_ref, a_ref) in ((s1_ref, X1_ref, r1_ref, a1_ref),
                                         (s2_ref, X2_ref, r2_ref, a2_ref)):
        s = s_ref[...]
        ex_self = X_ref[...]
        tot = s[:NPAD] + s[NPAD:] + ex_self
        r = 1.0 / (tot + 1e-16)
        r_ref[...] = r
        a_ref[...] = ex_self * r


def _tc_mid(s1, X1p, s2, X2p):
    out = jax.ShapeDtypeStruct((NPAD, 16), _f32)
    return pl.pallas_call(
        _tc_mid_body,
        out_shape=[out, out, out, out],
    )(s1, X1p, s2, X2p)


K2 = 256
NCHUNK2 = PER_W // K2


@functools.partial(
    pl.kernel,
    out_type=[
        jax.ShapeDtypeStruct((NC * NPAD, 64), _f32),
        jax.ShapeDtypeStruct((NC * NPAD, 64), _f32),
    ],
    mesh=_SC_MESH,
    compiler_params=pltpu.CompilerParams(use_tc_tiling_on_sc=False, needs_layout_passes=False),
    scratch_types=[
        pltpu.VMEM((K2,), _i32), pltpu.VMEM((K2,), _i32),
        pltpu.VMEM((K2,), _i32), pltpu.VMEM((K2,), _i32),
        pltpu.VMEM((K2, 64), _f32), pltpu.VMEM((K2, 64), _f32),
        pltpu.VMEM((K2, 16), _f32), pltpu.VMEM((K2, 16), _f32),
        pltpu.VMEM((K2, 16), _f32), pltpu.VMEM((K2, 16), _f32),
        pltpu.VMEM((K2, 64), _f32), pltpu.VMEM((K2, 64), _f32),
        pltpu.VMEM((K2,), _i32), pltpu.VMEM((K2,), _i32),
        pltpu.VMEM_SHARED((NPAD, 64), _f32),
        pltpu.SemaphoreType.DMA,
        pltpu.SemaphoreType.DMA,
        pltpu.SemaphoreType.DMA,
        pltpu.SemaphoreType.DMA,
    ],
)
def _sc_pass2_12(src1, dst1, src2, dst2, H1, H2, ex1, ex2, r1, r2,
                 o1, o2,
                 idx_s0, idx_s1, idx_d0, idx_d1, hrows0, hrows1,
                 exrows0, exrows1, rrows0, rrows1, stage0, stage1,
                 sidx0, sidx1, acc,
                 sem0, sem1, sem_s0, sem_s1):
    cid = lax.axis_index("c")
    sid = lax.axis_index("s")
    wid = sid * NC + cid

    idx_s = (idx_s0, idx_s1)
    idx_d = (idx_d0, idx_d1)
    hrows = (hrows0, hrows1)
    exrows = (exrows0, exrows1)
    rrows = (rrows0, rrows1)
    stage = (stage0, stage1)
    sidx = (sidx0, sidx1)
    sem = (sem0, sem1)
    sem_s = (sem_s0, sem_s1)

    zero16 = jnp.zeros((16,), _f32)
    rs = pl.ds(sid * RPT, RPT)

    for (src, dst, H, exi, r, oo) in ((src1, dst1, H1, ex1, r1, o1),
                                      (src2, dst2, H2, ex2, r2, o2)):
        def load_idx(j, b):
            base = wid * PER_W + j * K2
            pltpu.sync_copy(src.at[pl.ds(base, K2)], idx_s[b])
            pltpu.sync_copy(dst.at[pl.ds(base, K2)], idx_d[b])

        def fire(j, b):
            base = wid * PER_W + j * K2
            pltpu.async_copy(H.at[idx_s[b]], hrows[b], sem[b])
            pltpu.async_copy(r.at[idx_d[b]], rrows[b], sem[b])
            pltpu.async_copy(exi.at[pl.ds(base, K2)], exrows[b], sem[b])

        def wait(j, b):
            base = wid * PER_W + j * K2
            pltpu.make_async_copy(H.at[idx_s[b]], hrows[b], sem[b]).wait()
            pltpu.make_async_copy(r.at[idx_d[b]], rrows[b], sem[b]).wait()
            pltpu.make_async_copy(exi.at[pl.ds(base, K2)], exrows[b], sem[b]).wait()

        @pl.loop(0, K2)
        def _zero(i):
            for q in range(4):
                stage0[i, pl.ds(q * 16, 16)] = zero16

        off = 0
        while off < RPT:
            n = min(K2, RPT - off)
            pltpu.sync_copy(stage0.at[pl.ds(0, n)],
                            acc.at[pl.ds(sid * RPT + off, n)])
            off += n
        load_idx(0, 0)
        fire(0, 0)
        plsc.subcore_barrier()

        @pl.loop(0, NCHUNK2 // 2)
        def _pair(j2):
            for b in (0, 1):
                j = j2 * 2 + b
                nb = 1 - b
                nxt = j + 1

                @pl.when(nxt < NCHUNK2)
                def _prefetch():
                    load_idx(nxt, nb)
                    fire(nxt, nb)

                wait(j, b)

                @pl.when(j >= 2)
                def _drain():
                    pltpu.make_async_copy(stage[b], acc.at[sidx[b]], sem_s[b]).wait()

                h_b, ex_b, r_b, st_b = hrows[b], exrows[b], rrows[b], stage[b]

                @pl.loop(0, K2, unroll=4)
                def _edge(i):
                    alpha = ex_b[i, :] * r_b[i, :]
                    for q in range(4):
                        st_b[i, pl.ds(q * 16, 16)] = h_b[i, pl.ds(q * 16, 16)] * alpha

                id_b, si_b = idx_d[b], sidx[b]

                @pl.loop(0, K2 // 16)
                def _sicopy(g):
                    si_b[pl.ds(g * 16, 16)] = id_b[pl.ds(g * 16, 16)]

                pltpu.async_copy(st_b, acc.at[sidx[b]], sem_s[b], add=True)

        for b in (NCHUNK2 % 2, (NCHUNK2 - 1) % 2):
            pltpu.make_async_copy(stage[b], acc.at[sidx[b]], sem_s[b]).wait()

        plsc.subcore_barrier()
        pltpu.sync_copy(acc.at[rs], oo.at[pl.ds(cid * NPAD + sid * RPT, RPT)])
        plsc.subcore_barrier()


def _tc_layer3_body(o1a_ref, o1b_ref, H1_ref, a1_ref, b1_ref,
                    o2a_ref, o2b_ref, H2_ref, a2_ref, b2_ref,
                    W3_ref, as3_ref, ad3_ref,
                    H3_ref, A3_ref):
    xs = []
    for (oa, ob, H_ref, a_ref, b_ref) in ((o1a_ref, o1b_ref, H1_ref, a1_ref, b1_ref),
                                          (o2a_ref, o2b_ref, H2_ref, a2_ref, b2_ref)):
        aself = a_ref[...][:, :8]
        xl = oa[...] + ob[...] + H_ref[...] * jnp.tile(aself, (1, 8)) + b_ref[...]
        xl = jnp.where(xl > 0, xl, jnp.exp(jnp.minimum(xl, 0.0)) - 1.0)
        xs.append(xl)
    xc = jnp.concatenate(xs, axis=1)
    h3 = jnp.dot(xc, W3_ref[...], preferred_element_type=_f32)
    a3s = jnp.sum(h3 * as3_ref[...], axis=1, keepdims=True)
    a3d = jnp.sum(h3 * ad3_ref[...], axis=1, keepdims=True)
    t = a3s + a3d
    ex_self = jnp.exp(jnp.maximum(t, 0.2 * t))
    H3_ref[...] = h3
    A3_ref[...] = jnp.concatenate(
        [a3s, a3d, ex_self, jnp.zeros_like(h3[:, :13])], axis=1)


def _tc_layer3(o1a, o1b, H1, a1, b1p, o2a, o2b, H2, a2, b2p, W3r, as3, ad3):
    R = 1000
    grid = (N // R,)
    row = lambda i: (i, 0)
    const = lambda i: (0, 0)
    return pl.pallas_call(
        _tc_layer3_body,
        grid=grid,
        in_specs=[
            pl.BlockSpec((R, 64), row), pl.BlockSpec((R, 64), row),
            pl.BlockSpec((R, 64), row), pl.BlockSpec((R, 16), row),
            pl.BlockSpec((1, 64), const),
            pl.BlockSpec((R, 64), row), pl.BlockSpec((R, 64), row),
            pl.BlockSpec((R, 64), row), pl.BlockSpec((R, 16), row),
            pl.BlockSpec((1, 64), const),
            pl.BlockSpec((D, C), const),
            pl.BlockSpec((1, C), const), pl.BlockSpec((1, C), const),
        ],
        out_specs=[pl.BlockSpec((R, C), row), pl.BlockSpec((R, 16), row)],
        out_shape=[jax.ShapeDtypeStruct((N, C), _f32),
                   jax.ShapeDtypeStruct((N, 16), _f32)],
    )(o1a, o1b, H1, a1, b1p, o2a, o2b, H2, a2, b2p, W3r, as3, ad3)


@functools.partial(
    pl.kernel,
    out_type=[
        jax.ShapeDtypeStruct((EPAD,), _f32),
        jax.ShapeDtypeStruct((NC * NPAD,), _f32),
    ],
    mesh=_SC_MESH,
    compiler_params=pltpu.CompilerParams(use_tc_tiling_on_sc=False, needs_layout_passes=False),
    scratch_types=[
        pltpu.VMEM((NPAD,), _f32),
        pltpu.VMEM((NPAD,), _f32),
        pltpu.VMEM((K,), _i32),
        pltpu.VMEM((K,), _i32),
        pltpu.VMEM((K,), _f32),
        pltpu.VMEM_SHARED((NPAD,), _f32),
        pltpu.SemaphoreType.DMA,
    ],
)
def _sc_pass1_3(src1, dst1, a3s_t, a3d_t,
                ex3, s3o,
                a3s_l, a3d_l, idx_s, idx_d, exst, acc3, sem1):
    cid = lax.axis_index("c")
    sid = lax.axis_index("s")
    wid = sid * NC + cid

    pltpu.sync_copy(a3s_t, a3s_l)
    pltpu.sync_copy(a3d_t, a3d_l)

    zero16 = jnp.zeros((16,), _f32)

    @pl.loop(0, K // 16)
    def _zero(g):
        exst[pl.ds(g * 16, 16)] = zero16

    pltpu.sync_copy(exst.at[pl.ds(0, RPT)], acc3.at[pl.ds(sid * RPT, RPT)])
    plsc.subcore_barrier()

    @pl.loop(0, NCHUNK)
    def _chunk(j):
        base = wid * PER_W + j * K
        pltpu.sync_copy(src1.at[pl.ds(base, K)], idx_s)
        pltpu.sync_copy(dst1.at[pl.ds(base, K)], idx_d)

        @pl.loop(0, K // 16, unroll=2)
        def _grp(g):
            sv = idx_s[pl.ds(g * 16, 16)]
            dv = idx_d[pl.ds(g * 16, 16)]
            av = plsc.load_gather(a3s_l, [sv])
            bv = plsc.load_gather(a3d_l, [dv])
            v = av + bv
            exst[pl.ds(g * 16, 16)] = jnp.exp(jnp.maximum(v, 0.2 * v))

        pltpu.sync_copy(exst, ex3.at[pl.ds(base, K)])
        pltpu.sync_copy(exst, acc3.at[idx_d], add=True)

    plsc.subcore_barrier()
    rs = pl.ds(sid * RPT, RPT)
    pltpu.sync_copy(acc3.at[rs], s3o.at[pl.ds(cid * NPAD + sid * RPT, RPT)])


def _tc_mid3_body(s3_ref, X3_ref, r3_ref, a3_ref):
    s = s3_ref[...]
    ex_self = X3_ref[...]
    sa = s[: (NPAD // 16)]
    sb = s[(NPAD // 16):]
    r = 1.0 / (sa + sb + ex_self + 1e-16)
    r3_ref[...] = r
    a3_ref[...] = ex_self * r


def _tc_mid3(s3r, X3r):
    out = jax.ShapeDtypeStruct((NPAD // 16, 16), _f32)
    return pl.pallas_call(
        _tc_mid3_body,
        out_shape=[out, out],
    )(s3r, X3r)


@functools.partial(
    pl.kernel,
    out_type=[
        jax.ShapeDtypeStruct((NC * NPAD, 16), _f32),
    ],
    mesh=_SC_MESH,
    compiler_params=pltpu.CompilerParams(use_tc_tiling_on_sc=False, needs_layout_passes=False),
    scratch_types=[
        pltpu.VMEM((NPAD,), _f32),
        pltpu.VMEM((K,), _i32), pltpu.VMEM((K,), _i32),
        pltpu.VMEM((K,), _i32), pltpu.VMEM((K,), _i32),
        pltpu.VMEM((K, 16), _f32), pltpu.VMEM((K, 16), _f32),
        pltpu.VMEM((K,), _f32), pltpu.VMEM((K,), _f32),
        pltpu.VMEM((K, 16), _f32), pltpu.VMEM((K, 16), _f32),
        pltpu.VMEM((K,), _i32), pltpu.VMEM((K,), _i32),
        pltpu.VMEM_SHARED((NPAD, 16), _f32),
        pltpu.SemaphoreType.DMA,
        pltpu.SemaphoreType.DMA,
        pltpu.SemaphoreType.DMA,
        pltpu.SemaphoreType.DMA,
    ],
)
def _sc_pass2_3(src1, dst1, H3, ex3, r3_t,
                o3,
                r3_l, idx_s0, idx_s1, idx_d0, idx_d1, h3rows0, h3rows1,
                exc0, exc1, stage0, stage1, sidx0, sidx1, acc,
                sem0, sem1, sem_s0, sem_s1):
    cid = lax.axis_index("c")
    sid = lax.axis_index("s")
    wid = sid * NC + cid

    idx_s = (idx_s0, idx_s1)
    idx_d = (idx_d0, idx_d1)
    h3rows = (h3rows0, h3rows1)
    exc = (exc0, exc1)
    stage = (stage0, stage1)
    sidx = (sidx0, sidx1)
    sem = (sem0, sem1)
    sem_s = (sem_s0, sem_s1)

    pltpu.sync_copy(r3_t, r3_l)

    zero16 = jnp.zeros((16,), _f32)

    @pl.loop(0, K)
    def _zero(i):
        stage0[i, :] = zero16

    def load_idx(j, b):
        base = wid * PER_W + j * K
        pltpu.sync_copy(src1.at[pl.ds(base, K)], idx_s[b])
        pltpu.sync_copy(dst1.at[pl.ds(base, K)], idx_d[b])

    def fire(j, b):
        base = wid * PER_W + j * K
        pltpu.async_copy(H3.at[idx_s[b]], h3rows[b], sem[b])
        pltpu.async_copy(ex3.at[pl.ds(base, K)], exc[b], sem[b])

    def wait(j, b):
        base = wid * PER_W + j * K
        pltpu.make_async_copy(H3.at[idx_s[b]], h3rows[b], sem[b]).wait()
        pltpu.make_async_copy(ex3.at[pl.ds(base, K)], exc[b], sem[b]).wait()

    pltpu.sync_copy(stage0.at[pl.ds(0, RPT)], acc.at[pl.ds(sid * RPT, RPT)])
    load_idx(0, 0)
    fire(0, 0)
    plsc.subcore_barrier()

    @pl.loop(0, NCHUNK // 2)
    def _pair(j2):
        for b in (0, 1):
            j = j2 * 2 + b
            nb = 1 - b
            nxt = j + 1

            @pl.when(nxt < NCHUNK)
            def _prefetch():
                load_idx(nxt, nb)
                fire(nxt, nb)

            wait(j, b)

            @pl.when(j >= 2)
            def _drain():
                pltpu.make_async_copy(stage[b], acc.at[sidx[b]], sem_s[b]).wait()

            h_b, ex_b, st_b, id_b = h3rows[b], exc[b], stage[b], idx_d[b]

            @pl.loop(0, K // 16, unroll=2)
            def _grp(g):
                b16 = pl.ds(g * 16, 16)
                dv = id_b[b16]
                rv = plsc.load_gather(r3_l, [dv])
                alpha = ex_b[b16] * rv
                for e in range(16):
                    row = g * 16 + e
                    st_b[row, :] = h_b[row, :] * alpha[e]

            si_b = sidx[b]

            @pl.loop(0, K // 16)
            def _sicopy(g):
                si_b[pl.ds(g * 16, 16)] = id_b[pl.ds(g * 16, 16)]

            pltpu.async_copy(st_b, acc.at[sidx[b]], sem_s[b], add=True)

    for b in (NCHUNK % 2, (NCHUNK - 1) % 2):
        pltpu.make_async_copy(stage[b], acc.at[sidx[b]], sem_s[b]).wait()

    plsc.subcore_barrier()
    rs = pl.ds(sid * RPT, RPT)
    pltpu.sync_copy(acc.at[rs], o3.at[pl.ds(cid * NPAD + sid * RPT, RPT)])


def _tc_final_body(o3a_ref, o3b_ref, H3_ref, a3_ref, b3_ref, out_ref):
    z = (o3a_ref[...] + o3b_ref[...]
         + H3_ref[...] * a3_ref[...]
         + b3_ref[...])
    m = jnp.max(z, axis=1, keepdims=True)
    zm = z - m
    out_ref[...] = zm - jnp.log(jnp.sum(jnp.exp(zm), axis=1, keepdims=True))


def _tc_final(o3a, o3b, H3, aself3, b3):
    R = 1000
    grid = (N // R,)
    row = lambda i: (i, 0)
    const = lambda i: (0, 0)
    return pl.pallas_call(
        _tc_final_body,
        grid=grid,
        in_specs=[
            pl.BlockSpec((R, C), row), pl.BlockSpec((R, C), row),
            pl.BlockSpec((R, C), row), pl.BlockSpec((R, 1), row),
            pl.BlockSpec((1, C), const),
        ],
        out_specs=pl.BlockSpec((R, C), row),
        out_shape=jax.ShapeDtypeStruct((N, C), _f32),
    )(o3a, o3b, H3, aself3, b3)


def _build_B(att):
    rows = np.arange(64)
    cols = rows % 8
    return jnp.zeros((64, 8), _f32).at[rows, cols].set(att.T.reshape(64))


def _pad_edges(ei):
    npd = EPAD - E
    pad_src = jnp.zeros((npd,), _i32)
    pad_dst = jnp.asarray(N + (np.arange(npd) % PADROWS), _i32)
    src = jnp.concatenate([ei[0], pad_src])
    dst = jnp.concatenate([ei[1], pad_dst])
    return src, dst


def kernel(x, edge_index, topo_edges, W1, as1, ad1, b1, W2, as2, ad2, b2,
           W3, as3, ad3, b3):
    W1p = W1[:, _PERM]
    W2p = W2[:, _PERM]
    B1s, B1d = _build_B(as1), _build_B(ad1)
    B2s, B2d = _build_B(as2), _build_B(ad2)
    b1p = b1[_PERM][None, :]
    b2p = b2[_PERM][None, :]
    orig_perm = np.concatenate([_ORIG, _ORIG + 64])
    W3r = W3[orig_perm, :]

    src1, dst1 = _pad_edges(edge_index)
    src2, dst2 = _pad_edges(topo_edges)

    H1, S1, D1, X1, H2, S2, D2, X2 = _tc_prep12(x, W1p, B1s, B1d, W2p, B2s, B2d)

    zpad16 = jnp.zeros((NPAD - N, 16), _f32)
    npad16 = jnp.full((NPAD - N, 16), -1e30, _f32)
    S1p = jnp.concatenate([S1, zpad16])
    D1p = jnp.concatenate([D1, npad16])
    S2p = jnp.concatenate([S2, zpad16])
    D2p = jnp.concatenate([D2, npad16])
    X1p = jnp.concatenate([X1, zpad16])
    X2p = jnp.concatenate([X2, zpad16])

    ex1, ex2, s1, s2 = _sc_pass1_12(src1, dst1, src2, dst2, S1p, D1p, S2p, D2p)

    r1, a1self, r2, a2self = _tc_mid(s1, X1p, s2, X2p)

    o1, o2 = _sc_pass2_12(src1, dst1, src2, dst2, H1, H2, ex1, ex2, r1, r2)

    H3, A3 = _tc_layer3(o1[:N], o1[NPAD:NPAD + N], H1, a1self[:N], b1p,
                        o2[:N], o2[NPAD:NPAD + N], H2, a2self[:N], b2p,
                        W3r, as3, ad3)

    zpad1 = jnp.zeros((NPAD - N,), _f32)
    a3s_t = jnp.concatenate([A3[:, 0], zpad1])
    a3d_t = jnp.concatenate([A3[:, 1], jnp.full((NPAD - N,), -1e30, _f32)])
    X3p = jnp.concatenate([A3[:, 2], zpad1])

    ex3, s3 = _sc_pass1_3(src1, dst1, a3s_t, a3d_t)

    r3, a3self = _tc_mid3(s3.reshape(2 * NPAD // 16, 16), X3p.reshape(NPAD // 16, 16))

    (o3,) = _sc_pass2_3(src1, dst1, H3, ex3, r3.reshape(-1))

    out = _tc_final(o3[:N], o3[NPAD:NPAD + N], H3,
                    a3self.reshape(-1)[:N, None], b3[None, :])
    return out


# speedup vs baseline: 1.0468x; 1.0050x over previous
import functools

import numpy as np
import jax
import jax.numpy as jnp
from jax import lax
from jax.experimental import pallas as pl
from jax.experimental.pallas import tpu as pltpu
from jax.experimental.pallas import tpu_sc as plsc

N = 10000
E = 320000
D = 128
C = 16

NC = 2
NS = 16
NW = NC * NS
K = 1024
PER_W = 10240
EPAD = NW * PER_W
NCHUNK = PER_W // K
NPAD = 10112
RPT = NPAD // NS
PADROWS = NPAD - N

_f32 = jnp.float32
_i32 = jnp.int32

_SC_MESH = plsc.VectorSubcoreMesh(core_axis_name="c", subcore_axis_name="s")

_PERM = np.array([(p % 8) * 8 + p // 8 for p in range(64)])
_ORIG = np.array([(p % 8) * 8 + p // 8 for p in range(64)])


def _tc_prep12_body(x_ref, W1_ref, B1s_ref, B1d_ref, W2_ref, B2s_ref, B2d_ref,
                    H1_ref, S1_ref, D1_ref, X1_ref, H2_ref, S2_ref, D2_ref, X2_ref):
    xb = x_ref[...]
    for (W_ref, Bs_ref, Bd_ref, H_ref, S_ref, D_ref, X_ref) in (
        (W1_ref, B1s_ref, B1d_ref, H1_ref, S1_ref, D1_ref, X1_ref),
        (W2_ref, B2s_ref, B2d_ref, H2_ref, S2_ref, D2_ref, X2_ref),
    ):
        h = jnp.dot(xb, W_ref[...], preferred_element_type=_f32)
        a_s = jnp.dot(h, Bs_ref[...], preferred_element_type=_f32)
        a_d = jnp.dot(h, Bd_ref[...], preferred_element_type=_f32)
        H_ref[...] = h
        S_ref[...] = jnp.concatenate([a_s, a_s], axis=1)
        D_ref[...] = jnp.concatenate([a_d, a_d], axis=1)
        t = a_s + a_d
        ex = jnp.exp(jnp.maximum(t, 0.2 * t))
        X_ref[...] = jnp.concatenate([ex, ex], axis=1)


def _tc_prep12(x, W1p, B1s, B1d, W2p, B2s, B2d):
    R = 1000
    grid = (N // R,)
    row = lambda i: (i, 0)
    const = lambda i: (0, 0)
    out16 = jax.ShapeDtypeStruct((N, 16), _f32)
    out8 = jax.ShapeDtypeStruct((N, 8), _f32)
    return pl.pallas_call(
        _tc_prep12_body,
        grid=grid,
        in_specs=[
            pl.BlockSpec((R, D), row),
            pl.BlockSpec((D, 64), const), pl.BlockSpec((64, 8), const), pl.BlockSpec((64, 8), const),
            pl.BlockSpec((D, 64), const), pl.BlockSpec((64, 8), const), pl.BlockSpec((64, 8), const),
        ],
        out_specs=[
            pl.BlockSpec((R, 64), row), pl.BlockSpec((R, 16), row),
            pl.BlockSpec((R, 16), row), pl.BlockSpec((R, 16), row),
            pl.BlockSpec((R, 64), row), pl.BlockSpec((R, 16), row),
            pl.BlockSpec((R, 16), row), pl.BlockSpec((R, 16), row),
        ],
        out_shape=[jax.ShapeDtypeStruct((N, 64), _f32), out16, out16, out16,
                   jax.ShapeDtypeStruct((N, 64), _f32), out16, out16, out16],
    )(x, W1p, B1s, B1d, W2p, B2s, B2d)


@functools.partial(
    pl.kernel,
    out_type=[
        jax.ShapeDtypeStruct((EPAD, 16), _f32),
        jax.ShapeDtypeStruct((EPAD, 16), _f32),
        jax.ShapeDtypeStruct((NC * NPAD, 16), _f32),
        jax.ShapeDtypeStruct((NC * NPAD, 16), _f32),
    ],
    mesh=_SC_MESH,
    compiler_params=pltpu.CompilerParams(use_tc_tiling_on_sc=False, needs_layout_passes=False),
    scratch_types=[
        pltpu.VMEM((K,), _i32), pltpu.VMEM((K,), _i32),
        pltpu.VMEM((K,), _i32), pltpu.VMEM((K,), _i32),
        pltpu.VMEM((K, 16), _f32), pltpu.VMEM((K, 16), _f32),
        pltpu.VMEM((K, 16), _f32), pltpu.VMEM((K, 16), _f32),
        pltpu.VMEM((K, 16), _f32), pltpu.VMEM((K, 16), _f32),
        pltpu.VMEM((K,), _i32), pltpu.VMEM((K,), _i32),
        pltpu.VMEM_SHARED((NPAD, 16), _f32),
        pltpu.VMEM_SHARED((NPAD, 16), _f32),
        pltpu.SemaphoreType.DMA,
        pltpu.SemaphoreType.DMA,
        pltpu.SemaphoreType.DMA,
        pltpu.SemaphoreType.DMA,
        pltpu.SemaphoreType.DMA,
        pltpu.SemaphoreType.DMA,
    ],
)
def _sc_pass1_12(src1, dst1, src2, dst2, S1, D1, S2, D2,
                 ex1, ex2, s1o, s2o,
                 idx_s0, idx_s1, idx_d0, idx_d1, rows_s0, rows_s1,
                 rows_d0, rows_d1, stage0, stage1,
                 sidx0, sidx1, acc1, acc2,
                 sem0, sem1, sem_s0, sem_s1, sem_e0, sem_e1):
    cid = lax.axis_index("c")
    sid = lax.axis_index("s")
    wid = sid * NC + cid

    idx_s = (idx_s0, idx_s1)
    idx_d = (idx_d0, idx_d1)
    rows_s = (rows_s0, rows_s1)
    rows_d = (rows_d0, rows_d1)
    stage = (stage0, stage1)
    sidx = (sidx0, sidx1)
    sem = (sem0, sem1)
    sem_s = (sem_s0, sem_s1)
    sem_e = (sem_e0, sem_e1)

    zero16 = jnp.zeros((16,), _f32)

    @pl.loop(0, K)
    def _zero(i):
        stage0[i, :] = zero16

    for acc in (acc1, acc2):
        pltpu.sync_copy(stage0.at[pl.ds(0, RPT)], acc.at[pl.ds(sid * RPT, RPT)])
    plsc.subcore_barrier()

    for (src, dst, S, Dt, exo, acc) in ((src1, dst1, S1, D1, ex1, acc1),
                                        (src2, dst2, S2, D2, ex2, acc2)):
        def load_idx(j, b):
            base = wid * PER_W + j * K
            pltpu.sync_copy(src.at[pl.ds(base, K)], idx_s[b])
            pltpu.sync_copy(dst.at[pl.ds(base, K)], idx_d[b])

        def fire(b):
            pltpu.async_copy(S.at[idx_s[b]], rows_s[b], sem[b])
            pltpu.async_copy(Dt.at[idx_d[b]], rows_d[b], sem[b])

        def wait(b):
            pltpu.make_async_copy(S.at[idx_s[b]], rows_s[b], sem[b]).wait()
            pltpu.make_async_copy(Dt.at[idx_d[b]], rows_d[b], sem[b]).wait()

        load_idx(0, 0)
        fire(0)

        @pl.loop(0, NCHUNK // 2)
        def _pair(j2):
            for b in (0, 1):
                j = j2 * 2 + b
                nb = 1 - b
                nxt = j + 1

                @pl.when(nxt < NCHUNK)
                def _prefetch():
                    load_idx(nxt, nb)
                    fire(nb)

                wait(b)

                @pl.when(j >= 2)
                def _drain():
                    base_p = wid * PER_W + (j - 2) * K
                    pltpu.make_async_copy(stage[b], exo.at[pl.ds(base_p, K)], sem_e[b]).wait()
                    pltpu.make_async_copy(stage[b], acc.at[sidx[b]], sem_s[b]).wait()

                rs_b, rd_b, st_b = rows_s[b], rows_d[b], stage[b]

                @pl.loop(0, K, unroll=8)
                def _edge(i):
                    v = rs_b[i, :] + rd_b[i, :]
                    st_b[i, :] = jnp.exp(jnp.maximum(v, 0.2 * v))

                base = wid * PER_W + j * K
                id_b, si_b = idx_d[b], sidx[b]

                @pl.loop(0, K // 16)
                def _sicopy(g):
                    si_b[pl.ds(g * 16, 16)] = id_b[pl.ds(g * 16, 16)]

                pltpu.async_copy(st_b, exo.at[pl.ds(base, K)], sem_e[b])
                pltpu.async_copy(st_b, acc.at[sidx[b]], sem_s[b], add=True)

        for b, j_last in ((NCHUNK % 2, NCHUNK - 2), ((NCHUNK - 1) % 2, NCHUNK - 1)):
            base_p = wid * PER_W + j_last * K
            pltpu.make_async_copy(stage[b], exo.at[pl.ds(base_p, K)], sem_e[b]).wait()
            pltpu.make_async_copy(stage[b], acc.at[sidx[b]], sem_s[b]).wait()

    plsc.subcore_barrier()
    rs = pl.ds(sid * RPT, RPT)
    pltpu.sync_copy(acc1.at[rs], s1o.at[pl.ds(cid * NPAD + sid * RPT, RPT)])
    pltpu.sync_copy(acc2.at[rs], s2o.at[pl.ds(cid * NPAD + sid * RPT, RPT)])


K2 = 256
NCHUNK2 = PER_W // K2


@functools.partial(
    pl.kernel,
    out_type=[
        jax.ShapeDtypeStruct((NC * NPAD, 64), _f32),
        jax.ShapeDtypeStruct((NC * NPAD, 64), _f32),
    ],
    mesh=_SC_MESH,
    compiler_params=pltpu.CompilerParams(use_tc_tiling_on_sc=False, needs_layout_passes=False),
    scratch_types=[
        pltpu.VMEM((K2,), _i32), pltpu.VMEM((K2,), _i32),
        pltpu.VMEM((K2,), _i32), pltpu.VMEM((K2,), _i32),
        pltpu.VMEM((K2, 64), _f32), pltpu.VMEM((K2, 64), _f32),
        pltpu.VMEM((K2, 16), _f32), pltpu.VMEM((K2, 16), _f32),
        pltpu.VMEM((K2, 64), _f32), pltpu.VMEM((K2, 64), _f32),
        pltpu.VMEM((K2,), _i32), pltpu.VMEM((K2,), _i32),
        pltpu.VMEM_SHARED((NPAD, 64), _f32),
        pltpu.SemaphoreType.DMA,
        pltpu.SemaphoreType.DMA,
        pltpu.SemaphoreType.DMA,
        pltpu.SemaphoreType.DMA,
    ],
)
def _sc_pass2_12(src1, dst1, src2, dst2, H1, H2, ex1, ex2,
                 o1, o2,
                 idx_s0, idx_s1, idx_d0, idx_d1, hrows0, hrows1,
                 exrows0, exrows1, stage0, stage1,
                 sidx0, sidx1, acc,
                 sem0, sem1, sem_s0, sem_s1):
    cid = lax.axis_index("c")
    sid = lax.axis_index("s")
    wid = sid * NC + cid

    idx_s = (idx_s0, idx_s1)
    idx_d = (idx_d0, idx_d1)
    hrows = (hrows0, hrows1)
    exrows = (exrows0, exrows1)
    stage = (stage0, stage1)
    sidx = (sidx0, sidx1)
    sem = (sem0, sem1)
    sem_s = (sem_s0, sem_s1)

    zero16 = jnp.zeros((16,), _f32)
    rs = pl.ds(sid * RPT, RPT)

    for (src, dst, H, exi, oo) in ((src1, dst1, H1, ex1, o1),
                                   (src2, dst2, H2, ex2, o2)):
        def load_idx(j, b):
            base = wid * PER_W + j * K2
            pltpu.sync_copy(src.at[pl.ds(base, K2)], idx_s[b])
            pltpu.sync_copy(dst.at[pl.ds(base, K2)], idx_d[b])

        def fire(j, b):
            base = wid * PER_W + j * K2
            pltpu.async_copy(H.at[idx_s[b]], hrows[b], sem[b])
            pltpu.async_copy(exi.at[pl.ds(base, K2)], exrows[b], sem[b])

        def wait(j, b):
            base = wid * PER_W + j * K2
            pltpu.make_async_copy(H.at[idx_s[b]], hrows[b], sem[b]).wait()
            pltpu.make_async_copy(exi.at[pl.ds(base, K2)], exrows[b], sem[b]).wait()

        @pl.loop(0, K2)
        def _zero(i):
            for q in range(4):
                stage0[i, pl.ds(q * 16, 16)] = zero16

        off = 0
        while off < RPT:
            n = min(K2, RPT - off)
            pltpu.sync_copy(stage0.at[pl.ds(0, n)],
                            acc.at[pl.ds(sid * RPT + off, n)])
            off += n
        load_idx(0, 0)
        fire(0, 0)
        plsc.subcore_barrier()

        @pl.loop(0, NCHUNK2 // 2)
        def _pair(j2):
            for b in (0, 1):
                j = j2 * 2 + b
                nb = 1 - b
                nxt = j + 1

                @pl.when(nxt < NCHUNK2)
                def _prefetch():
                    load_idx(nxt, nb)
                    fire(nxt, nb)

                wait(j, b)

                @pl.when(j >= 2)
                def _drain():
                    pltpu.make_async_copy(stage[b], acc.at[sidx[b]], sem_s[b]).wait()

                h_b, ex_b, st_b = hrows[b], exrows[b], stage[b]

                @pl.loop(0, K2, unroll=4)
                def _edge(i):
                    alpha = ex_b[i, :]
                    for q in range(4):
                        st_b[i, pl.ds(q * 16, 16)] = h_b[i, pl.ds(q * 16, 16)] * alpha

                id_b, si_b = idx_d[b], sidx[b]

                @pl.loop(0, K2 // 16)
                def _sicopy(g):
                    si_b[pl.ds(g * 16, 16)] = id_b[pl.ds(g * 16, 16)]

                pltpu.async_copy(st_b, acc.at[sidx[b]], sem_s[b], add=True)

        for b in (NCHUNK2 % 2, (NCHUNK2 - 1) % 2):
            pltpu.make_async_copy(stage[b], acc.at[sidx[b]], sem_s[b]).wait()

        plsc.subcore_barrier()
        pltpu.sync_copy(acc.at[rs], oo.at[pl.ds(cid * NPAD + sid * RPT, RPT)])
        plsc.subcore_barrier()


def _tc_layer3_body(o1a_ref, o1b_ref, H1_ref, X1_ref, s1a_ref, s1b_ref, b1_ref,
                    o2a_ref, o2b_ref, H2_ref, X2_ref, s2a_ref, s2b_ref, b2_ref,
                    W3_ref, as3_ref, ad3_ref,
                    H3_ref, A3_ref):
    xs = []
    for (oa, ob, H_ref, X_ref, sa_ref, sb_ref, b_ref) in (
            (o1a_ref, o1b_ref, H1_ref, X1_ref, s1a_ref, s1b_ref, b1_ref),
            (o2a_ref, o2b_ref, H2_ref, X2_ref, s2a_ref, s2b_ref, b2_ref)):
        exs = X_ref[...][:, :8]
        r = 1.0 / (sa_ref[...][:, :8] + sb_ref[...][:, :8] + exs + 1e-16)
        num = oa[...] + ob[...] + H_ref[...] * jnp.tile(exs, (1, 8))
        xl = num * jnp.tile(r, (1, 8)) + b_ref[...]
        xl = jnp.where(xl > 0, xl, jnp.exp(jnp.minimum(xl, 0.0)) - 1.0)
        xs.append(xl)
    xc = jnp.concatenate(xs, axis=1)
    h3 = jnp.dot(xc, W3_ref[...], preferred_element_type=_f32)
    a3s = jnp.sum(h3 * as3_ref[...], axis=1, keepdims=True)
    a3d = jnp.sum(h3 * ad3_ref[...], axis=1, keepdims=True)
    t = a3s + a3d
    ex_self = jnp.exp(jnp.maximum(t, 0.2 * t))
    H3_ref[...] = h3
    A3_ref[...] = jnp.concatenate(
        [a3s, a3d, ex_self, jnp.zeros_like(h3[:, :13])], axis=1)


def _tc_layer3(o1a, o1b, H1, X1, s1a, s1b, b1p,
               o2a, o2b, H2, X2, s2a, s2b, b2p, W3r, as3, ad3):
    R = 1000
    grid = (N // R,)
    row = lambda i: (i, 0)
    const = lambda i: (0, 0)
    return pl.pallas_call(
        _tc_layer3_body,
        grid=grid,
        in_specs=[
            pl.BlockSpec((R, 64), row), pl.BlockSpec((R, 64), row),
            pl.BlockSpec((R, 64), row), pl.BlockSpec((R, 16), row),
            pl.BlockSpec((R, 16), row), pl.BlockSpec((R, 16), row),
            pl.BlockSpec((1, 64), const),
            pl.BlockSpec((R, 64), row), pl.BlockSpec((R, 64), row),
            pl.BlockSpec((R, 64), row), pl.BlockSpec((R, 16), row),
            pl.BlockSpec((R, 16), row), pl.BlockSpec((R, 16), row),
            pl.BlockSpec((1, 64), const),
            pl.BlockSpec((D, C), const),
            pl.BlockSpec((1, C), const), pl.BlockSpec((1, C), const),
        ],
        out_specs=[pl.BlockSpec((R, C), row), pl.BlockSpec((R, 16), row)],
        out_shape=[jax.ShapeDtypeStruct((N, C), _f32),
                   jax.ShapeDtypeStruct((N, 16), _f32)],
    )(o1a, o1b, H1, X1, s1a, s1b, b1p, o2a, o2b, H2, X2, s2a, s2b, b2p,
      W3r, as3, ad3)


@functools.partial(
    pl.kernel,
    out_type=[
        jax.ShapeDtypeStruct((EPAD,), _f32),
        jax.ShapeDtypeStruct((NC * NPAD,), _f32),
    ],
    mesh=_SC_MESH,
    compiler_params=pltpu.CompilerParams(use_tc_tiling_on_sc=False, needs_layout_passes=False),
    scratch_types=[
        pltpu.VMEM((NPAD,), _f32),
        pltpu.VMEM((NPAD,), _f32),
        pltpu.VMEM((K,), _i32),
        pltpu.VMEM((K,), _i32),
        pltpu.VMEM((K,), _f32),
        pltpu.VMEM_SHARED((NPAD,), _f32),
        pltpu.SemaphoreType.DMA,
    ],
)
def _sc_pass1_3(src1, dst1, a3s_t, a3d_t,
                ex3, s3o,
                a3s_l, a3d_l, idx_s, idx_d, exst, acc3, sem1):
    cid = lax.axis_index("c")
    sid = lax.axis_index("s")
    wid = sid * NC + cid

    pltpu.sync_copy(a3s_t, a3s_l)
    pltpu.sync_copy(a3d_t, a3d_l)

    zero16 = jnp.zeros((16,), _f32)

    @pl.loop(0, K // 16)
    def _zero(g):
        exst[pl.ds(g * 16, 16)] = zero16

    pltpu.sync_copy(exst.at[pl.ds(0, RPT)], acc3.at[pl.ds(sid * RPT, RPT)])
    plsc.subcore_barrier()

    @pl.loop(0, NCHUNK)
    def _chunk(j):
        base = wid * PER_W + j * K
        pltpu.sync_copy(src1.at[pl.ds(base, K)], idx_s)
        pltpu.sync_copy(dst1.at[pl.ds(base, K)], idx_d)

        @pl.loop(0, K // 16, unroll=2)
        def _grp(g):
            sv = idx_s[pl.ds(g * 16, 16)]
            dv = idx_d[pl.ds(g * 16, 16)]
            av = plsc.load_gather(a3s_l, [sv])
            bv = plsc.load_gather(a3d_l, [dv])
            v = av + bv
            exst[pl.ds(g * 16, 16)] = jnp.exp(jnp.maximum(v, 0.2 * v))

        pltpu.sync_copy(exst, ex3.at[pl.ds(base, K)])
        pltpu.sync_copy(exst, acc3.at[idx_d], add=True)

    plsc.subcore_barrier()
    rs = pl.ds(sid * RPT, RPT)
    pltpu.sync_copy(acc3.at[rs], s3o.at[pl.ds(cid * NPAD + sid * RPT, RPT)])


@functools.partial(
    pl.kernel,
    out_type=[
        jax.ShapeDtypeStruct((NC * NPAD, 16), _f32),
    ],
    mesh=_SC_MESH,
    compiler_params=pltpu.CompilerParams(use_tc_tiling_on_sc=False, needs_layout_passes=False),
    scratch_types=[
        pltpu.VMEM((K,), _i32), pltpu.VMEM((K,), _i32),
        pltpu.VMEM((K,), _i32), pltpu.VMEM((K,), _i32),
        pltpu.VMEM((K, 16), _f32), pltpu.VMEM((K, 16), _f32),
        pltpu.VMEM((K,), _f32), pltpu.VMEM((K,), _f32),
        pltpu.VMEM((K, 16), _f32), pltpu.VMEM((K, 16), _f32),
        pltpu.VMEM((K,), _i32), pltpu.VMEM((K,), _i32),
        pltpu.VMEM_SHARED((NPAD, 16), _f32),
        pltpu.SemaphoreType.DMA,
        pltpu.SemaphoreType.DMA,
        pltpu.SemaphoreType.DMA,
        pltpu.SemaphoreType.DMA,
    ],
)
def _sc_pass2_3(src1, dst1, H3, ex3,
                o3,
                idx_s0, idx_s1, idx_d0, idx_d1, h3rows0, h3rows1,
                exc0, exc1, stage0, stage1, sidx0, sidx1, acc,
                sem0, sem1, sem_s0, sem_s1):
    cid = lax.axis_index("c")
    sid = lax.axis_index("s")
    wid = sid * NC + cid

    idx_s = (idx_s0, idx_s1)
    idx_d = (idx_d0, idx_d1)
    h3rows = (h3rows0, h3rows1)
    exc = (exc0, exc1)
    stage = (stage0, stage1)
    sidx = (sidx0, sidx1)
    sem = (sem0, sem1)
    sem_s = (sem_s0, sem_s1)

    zero16 = jnp.zeros((16,), _f32)

    @pl.loop(0, K)
    def _zero(i):
        stage0[i, :] = zero16

    def load_idx(j, b):
        base = wid * PER_W + j * K
        pltpu.sync_copy(src1.at[pl.ds(base, K)], idx_s[b])
        pltpu.sync_copy(dst1.at[pl.ds(base, K)], idx_d[b])

    def fire(j, b):
        base = wid * PER_W + j * K
        pltpu.async_copy(H3.at[idx_s[b]], h3rows[b], sem[b])
        pltpu.async_copy(ex3.at[pl.ds(base, K)], exc[b], sem[b])

    def wait(j, b):
        base = wid * PER_W + j * K
        pltpu.make_async_copy(H3.at[idx_s[b]], h3rows[b], sem[b]).wait()
        pltpu.make_async_copy(ex3.at[pl.ds(base, K)], exc[b], sem[b]).wait()

    pltpu.sync_copy(stage0.at[pl.ds(0, RPT)], acc.at[pl.ds(sid * RPT, RPT)])
    load_idx(0, 0)
    fire(0, 0)
    plsc.subcore_barrier()

    @pl.loop(0, NCHUNK // 2)
    def _pair(j2):
        for b in (0, 1):
            j = j2 * 2 + b
            nb = 1 - b
            nxt = j + 1

            @pl.when(nxt < NCHUNK)
            def _prefetch():
                load_idx(nxt, nb)
                fire(nxt, nb)

            wait(j, b)

            @pl.when(j >= 2)
            def _drain():
                pltpu.make_async_copy(stage[b], acc.at[sidx[b]], sem_s[b]).wait()

            h_b, ex_b, st_b, id_b = h3rows[b], exc[b], stage[b], idx_d[b]

            @pl.loop(0, K // 16, unroll=2)
            def _grp(g):
                b16 = pl.ds(g * 16, 16)
                alpha = ex_b[b16]
                for e in range(16):
                    row = g * 16 + e
                    st_b[row, :] = h_b[row, :] * alpha[e]

            si_b = sidx[b]

            @pl.loop(0, K // 16)
            def _sicopy(g):
                si_b[pl.ds(g * 16, 16)] = id_b[pl.ds(g * 16, 16)]

            pltpu.async_copy(st_b, acc.at[sidx[b]], sem_s[b], add=True)

    for b in (NCHUNK % 2, (NCHUNK - 1) % 2):
        pltpu.make_async_copy(stage[b], acc.at[sidx[b]], sem_s[b]).wait()

    plsc.subcore_barrier()
    rs = pl.ds(sid * RPT, RPT)
    pltpu.sync_copy(acc.at[rs], o3.at[pl.ds(cid * NPAD + sid * RPT, RPT)])


def _tc_final_body(o3a_ref, o3b_ref, H3_ref, x3_ref, s3a_ref, s3b_ref,
                   b3_ref, out_ref):
    exs = x3_ref[...]
    r3 = 1.0 / (s3a_ref[...] + s3b_ref[...] + exs + 1e-16)
    z = (o3a_ref[...] + o3b_ref[...] + H3_ref[...] * exs) * r3 + b3_ref[...]
    m = jnp.max(z, axis=1, keepdims=True)
    zm = z - m
    out_ref[...] = zm - jnp.log(jnp.sum(jnp.exp(zm), axis=1, keepdims=True))


def _tc_final(o3a, o3b, H3, exs3, s3a, s3b, b3):
    R = 1000
    grid = (N // R,)
    row = lambda i: (i, 0)
    const = lambda i: (0, 0)
    return pl.pallas_call(
        _tc_final_body,
        grid=grid,
        in_specs=[
            pl.BlockSpec((R, C), row), pl.BlockSpec((R, C), row),
            pl.BlockSpec((R, C), row), pl.BlockSpec((R, 1), row),
            pl.BlockSpec((R, 1), row), pl.BlockSpec((R, 1), row),
            pl.BlockSpec((1, C), const),
        ],
        out_specs=pl.BlockSpec((R, C), row),
        out_shape=jax.ShapeDtypeStruct((N, C), _f32),
    )(o3a, o3b, H3, exs3, s3a, s3b, b3)


def _build_B(att):
    rows = np.arange(64)
    cols = rows % 8
    return jnp.zeros((64, 8), _f32).at[rows, cols].set(att.T.reshape(64))


def _pad_edges(ei):
    npd = EPAD - E
    pad_src = jnp.zeros((npd,), _i32)
    pad_dst = jnp.asarray(N + (np.arange(npd) % PADROWS), _i32)
    src = jnp.concatenate([ei[0], pad_src])
    dst = jnp.concatenate([ei[1], pad_dst])
    return src, dst


def kernel(x, edge_index, topo_edges, W1, as1, ad1, b1, W2, as2, ad2, b2,
           W3, as3, ad3, b3):
    W1p = W1[:, _PERM]
    W2p = W2[:, _PERM]
    B1s, B1d = _build_B(as1), _build_B(ad1)
    B2s, B2d = _build_B(as2), _build_B(ad2)
    b1p = b1[_PERM][None, :]
    b2p = b2[_PERM][None, :]
    orig_perm = np.concatenate([_ORIG, _ORIG + 64])
    W3r = W3[orig_perm, :]

    src1, dst1 = _pad_edges(edge_index)
    src2, dst2 = _pad_edges(topo_edges)

    H1, S1, D1, X1, H2, S2, D2, X2 = _tc_prep12(x, W1p, B1s, B1d, W2p, B2s, B2d)

    zpad16 = jnp.zeros((NPAD - N, 16), _f32)
    npad16 = jnp.full((NPAD - N, 16), -1e30, _f32)
    S1p = jnp.concatenate([S1, zpad16])
    D1p = jnp.concatenate([D1, npad16])
    S2p = jnp.concatenate([S2, zpad16])
    D2p = jnp.concatenate([D2, npad16])

    ex1, ex2, s1, s2 = _sc_pass1_12(src1, dst1, src2, dst2, S1p, D1p, S2p, D2p)

    o1, o2 = _sc_pass2_12(src1, dst1, src2, dst2, H1, H2, ex1, ex2)

    H3, A3 = _tc_layer3(o1[:N], o1[NPAD:NPAD + N], H1, X1,
                        s1[:N], s1[NPAD:NPAD + N], b1p,
                        o2[:N], o2[NPAD:NPAD + N], H2, X2,
                        s2[:N], s2[NPAD:NPAD + N], b2p,
                        W3r, as3, ad3)

    zpad1 = jnp.zeros((NPAD - N,), _f32)
    a3s_t = jnp.concatenate([A3[:, 0], zpad1])
    a3d_t = jnp.concatenate([A3[:, 1], jnp.full((NPAD - N,), -1e30, _f32)])

    ex3, s3 = _sc_pass1_3(src1, dst1, a3s_t, a3d_t)

    (o3,) = _sc_pass2_3(src1, dst1, H3, ex3)

    out = _tc_final(o3[:N], o3[NPAD:NPAD + N], H3, A3[:, 2][:, None],
                    s3[:N, None], s3[NPAD:NPAD + N, None], b3[None, :])
    return out
